# Initial kernel scaffold; baseline (speedup 1.0000x reference)
import jax
import jax.numpy as jnp
from jax.experimental import pallas as pl


def kernel(o_grid_x, d_grid_x, edge_index, edge_x, external_x, params):
    raise NotImplementedError("write your pallas kernel here")


# speedup vs baseline: 20.8349x; 20.8349x over previous
import functools

import jax
import jax.numpy as jnp
from jax import lax
from jax.experimental import pallas as pl
from jax.experimental.pallas import tpu as pltpu
from jax.experimental.pallas import tpu_sc as plsc

_B = 4096
_GRID = 25
_EPER = 48
_N = _B * _GRID
_F = 128

_RB = 200
_NBLK = _N // _RB
_KBLK = 22
_HEAD = _KBLK * _RB
_WIN = _RB + 48
_CPAD = 256
_HPR = 4448

_NW = 32
_RPW = 138
_CROWS = _NW * _RPW
_CH = 162
_EB = _CH * 2 * _EPER
_PRIV = _RPW * _CPAD


@functools.partial(
    pl.kernel,
    out_type=jax.ShapeDtypeStruct((_CROWS * _CPAD,), jnp.float32),
    mesh=plsc.VectorSubcoreMesh(core_axis_name="c", subcore_axis_name="s",
                                num_cores=2, num_subcores=16),
    scratch_types=[
        pltpu.VMEM((_EB,), jnp.int32),
        pltpu.VMEM((_PRIV,), jnp.float32),
    ],
    compiler_params=pltpu.CompilerParams(needs_layout_passes=False),
)
def _sc_band_counts(edge_ref, out_ref, edge_v, priv_v):
    cid = lax.axis_index("c")
    sid = lax.axis_index("s")
    w = sid * 2 + cid
    row0 = w * _RPW
    bstart = jnp.clip(row0 - 24, 0, _B - _CH)
    pltpu.sync_copy(edge_ref.at[pl.ds(bstart * (2 * _EPER), _EB)], edge_v)

    z16 = jnp.zeros((16,), jnp.float32)

    def _zero(i, c):
        priv_v[pl.ds(i * 16, 16)] = z16
        return c

    lax.fori_loop(0, _PRIV // 16, _zero, 0)

    lane = lax.iota(jnp.int32, 16)
    ones = jnp.ones((16,), jnp.float32)

    def _batch(bi, c):
        b = bstart + bi
        base = bi * (2 * _EPER)
        for j in range(_EPER // 16):
            sv = edge_v[pl.ds(base + j * 16, 16)]
            dv = edge_v[pl.ds(base + _EPER + j * 16, 16)]
            n = dv + b
            lrow = n - row0
            valid = (lrow >= 0) & (lrow < _RPW)
            blk = lax.div(n, _RB)
            jcol = (n - blk * _RB) + (sv - dv) + 24
            flat = jnp.where(valid, lrow * _CPAD + jcol, 0)
            for t in range(16):
                plsc.addupdate_scatter(priv_v, [flat], ones,
                                       mask=valid & (lane == t))
        return c

    lax.fori_loop(0, _CH, _batch, 0)
    pltpu.sync_copy(priv_v, out_ref.at[pl.ds(w * _PRIV, _PRIV)])


def _pass_body(xm_ref, xh_ref, w_ref, vec_ref, h_ref, st_ref, cent_ref):
    i = pl.program_id(0)
    x = jnp.where(i < _KBLK, xh_ref[...], xm_ref[...])
    scale = vec_ref[0, :]
    shift = vec_ref[1, :]
    arow = vec_ref[2, :]
    bias = vec_ref[3, :]
    y0 = x * scale[None, :] + shift[None, :]
    y = jnp.where(y0 >= 0.0, y0, arow[None, :] * y0)
    h = lax.dot_general(y, w_ref[...], (((1,), (1,)), ((), ())),
                        preferred_element_type=jnp.float32)
    h_ref[...] = h
    t = h + bias[None, :]

    @pl.when(i == 0)
    def _():
        st_ref[...] = jnp.zeros((8, _F), jnp.float32)

    st_ref[0:1, :] += jnp.sum(t, axis=0, keepdims=True)
    st_ref[1:2, :] += jnp.sum(t * t, axis=0, keepdims=True)
    cent_ref[...] = jnp.concatenate(
        [h[12 + 25 * q: 13 + 25 * q, :] for q in range(8)], axis=0)


def _gat_pass(xm, xh, w, vec):
    return pl.pallas_call(
        _pass_body,
        grid=(_NBLK,),
        in_specs=[
            pl.BlockSpec((_RB, _F), lambda i: (i, 0)),
            pl.BlockSpec((_RB, _F), lambda i: (jnp.minimum(i, _KBLK - 1), 0)),
            pl.BlockSpec((_F, _F), lambda i: (0, 0)),
            pl.BlockSpec((8, _F), lambda i: (0, 0)),
        ],
        out_specs=[
            pl.BlockSpec((_RB, _F), lambda i: (i, 0)),
            pl.BlockSpec((8, _F), lambda i: (0, 0)),
            pl.BlockSpec((8, _F), lambda i: (i, 0)),
        ],
        out_shape=[
            jax.ShapeDtypeStruct((_N, _F), jnp.float32),
            jax.ShapeDtypeStruct((8, _F), jnp.float32),
            jax.ShapeDtypeStruct((_B, _F), jnp.float32),
        ],
    )(xm, xh, w, vec)


def _band_body(cnt_ref, hp_ref, vec_ref, agg_ref, cent_ref, adj_ref):
    k = pl.program_id(0)
    hwin = hp_ref[pl.ds(k * _RB, _WIN), :]
    att_s = vec_ref[0, :]
    att_d = vec_ref[1, :]
    bias = vec_ref[2, :]
    asw = jnp.sum(hwin * att_s[None, :], axis=1)
    hblk = hwin[24:24 + _RB, :]
    ad = jnp.sum(hblk * att_d[None, :], axis=1)
    al = asw[None, :] + ad[:, None]
    al = jnp.where(al >= 0.0, al, 0.2 * al)
    cnt = cnt_ref[...][:, :_WIN]
    ri = lax.broadcasted_iota(jnp.int32, (_RB, _WIN), 0)
    ci = lax.broadcasted_iota(jnp.int32, (_RB, _WIN), 1)
    cnt = cnt + jnp.where(ci == ri + 24, 1.0, 0.0)
    pos = cnt > 0.0
    amax = jnp.max(jnp.where(pos, al, -1e30), axis=1)
    e = cnt * jnp.exp(al - amax[:, None])
    den = jnp.sum(e, axis=1)
    agg = lax.dot_general(e, hwin, (((1,), (0,)), ((), ())),
                          preferred_element_type=jnp.float32)
    agg = agg * (1.0 / (den + 1e-16))[:, None]
    agg_ref[...] = agg
    cent_ref[...] = jnp.concatenate(
        [agg[12 + 25 * q: 13 + 25 * q, :] for q in range(8)], axis=0)
    ta = agg + bias[None, :]
    th = hblk + bias[None, :]

    @pl.when(k == 0)
    def _():
        adj_ref[...] = jnp.zeros((8, _F), jnp.float32)

    adj_ref[0:1, :] += (jnp.sum(ta, axis=0, keepdims=True)
                        - jnp.sum(th, axis=0, keepdims=True))
    adj_ref[1:2, :] += (jnp.sum(ta * ta, axis=0, keepdims=True)
                        - jnp.sum(th * th, axis=0, keepdims=True))


def _band(counts, hpad, vec):
    return pl.pallas_call(
        _band_body,
        grid=(_KBLK,),
        in_specs=[
            pl.BlockSpec((_RB, _CPAD), lambda k: (k, 0)),
            pl.BlockSpec((_HPR, _F), lambda k: (0, 0)),
            pl.BlockSpec((8, _F), lambda k: (0, 0)),
        ],
        out_specs=[
            pl.BlockSpec((_RB, _F), lambda k: (k, 0)),
            pl.BlockSpec((8, _F), lambda k: (k, 0)),
            pl.BlockSpec((8, _F), lambda k: (0, 0)),
        ],
        out_shape=[
            jax.ShapeDtypeStruct((_HEAD, _F), jnp.float32),
            jax.ShapeDtypeStruct((_KBLK * 8, _F), jnp.float32),
            jax.ShapeDtypeStruct((8, _F), jnp.float32),
        ],
    )(counts, hpad, vec)


def _d0_body(ex_ref, xx_ref, we_ref, wx_ref, vb_ref, pe_ref, px_ref, st_ref):
    pe = lax.dot_general(ex_ref[...], we_ref[...], (((1,), (1,)), ((), ())),
                         preferred_element_type=jnp.float32)
    pe = pe + vb_ref[0:1, :64]
    px = lax.dot_general(xx_ref[...], wx_ref[...], (((1,), (1,)), ((), ())),
                         preferred_element_type=jnp.float32)
    px = px + vb_ref[1:2, :32]
    pe_ref[...] = pe
    px_ref[...] = px
    z64 = jnp.zeros((1, 64), jnp.float32)
    z96 = jnp.zeros((1, 96), jnp.float32)
    st_ref[...] = jnp.concatenate([
        jnp.concatenate([jnp.sum(pe, 0, keepdims=True), z64], axis=1),
        jnp.concatenate([jnp.sum(pe * pe, 0, keepdims=True), z64], axis=1),
        jnp.concatenate([jnp.sum(px, 0, keepdims=True), z96], axis=1),
        jnp.concatenate([jnp.sum(px * px, 0, keepdims=True), z96], axis=1),
        jnp.zeros((4, _F), jnp.float32),
    ], axis=0)


def _d0(edge_x, external_x, we, wx, vb):
    return pl.pallas_call(
        _d0_body,
        grid=(1,),
        in_specs=[
            pl.BlockSpec((_B, 32), lambda i: (0, 0)),
            pl.BlockSpec((_B, 16), lambda i: (0, 0)),
            pl.BlockSpec((64, 32), lambda i: (0, 0)),
            pl.BlockSpec((32, 16), lambda i: (0, 0)),
            pl.BlockSpec((8, _F), lambda i: (0, 0)),
        ],
        out_specs=[
            pl.BlockSpec((_B, 64), lambda i: (0, 0)),
            pl.BlockSpec((_B, 32), lambda i: (0, 0)),
            pl.BlockSpec((8, _F), lambda i: (0, 0)),
        ],
        out_shape=[
            jax.ShapeDtypeStruct((_B, 64), jnp.float32),
            jax.ShapeDtypeStruct((_B, 32), jnp.float32),
            jax.ShapeDtypeStruct((8, _F), jnp.float32),
        ],
    )(edge_x, external_x, we, wx, vb)


_DB = 256
_TPAD = 384
_HID = 256


def _d1_body(om_ref, oh_ref, dm_ref, dh_ref, pe_ref, px_ref, sv_ref, bv_ref,
             w_ref, h_ref, st_ref):
    i = pl.program_id(0)
    ri = lax.broadcasted_iota(jnp.int32, (_DB, 1), 0) + i * _DB
    use_h = ri < 176
    o_raw = jnp.where(use_h, oh_ref[...], om_ref[...])
    d_raw = jnp.where(use_h, dh_ref[...], dm_ref[...])
    tot = jnp.concatenate(
        [o_raw, d_raw, pe_ref[...], px_ref[...],
         jnp.zeros((_DB, 32), jnp.float32)], axis=1)
    scale = sv_ref[0, :]
    shift = sv_ref[1, :]
    arow = sv_ref[2, :]
    y0 = tot * scale[None, :] + shift[None, :]
    y = jnp.where(y0 >= 0.0, y0, arow[None, :] * y0)
    h = lax.dot_general(y, w_ref[...], (((1,), (1,)), ((), ())),
                        preferred_element_type=jnp.float32)
    h = h + bv_ref[0:1, :]
    h_ref[...] = h

    @pl.when(i == 0)
    def _():
        st_ref[...] = jnp.zeros((8, _HID), jnp.float32)

    st_ref[0:1, :] += jnp.sum(h, axis=0, keepdims=True)
    st_ref[1:2, :] += jnp.sum(h * h, axis=0, keepdims=True)


def _d1(om, oh, dm, dh, pe, px, sv, bv, w):
    return pl.pallas_call(
        _d1_body,
        grid=(_B // _DB,),
        in_specs=[
            pl.BlockSpec((_DB, _F), lambda i: (i, 0)),
            pl.BlockSpec((_DB, _F), lambda i: (0, 0)),
            pl.BlockSpec((_DB, _F), lambda i: (i, 0)),
            pl.BlockSpec((_DB, _F), lambda i: (0, 0)),
            pl.BlockSpec((_DB, 64), lambda i: (i, 0)),
            pl.BlockSpec((_DB, 32), lambda i: (i, 0)),
            pl.BlockSpec((8, _TPAD), lambda i: (0, 0)),
            pl.BlockSpec((8, _HID), lambda i: (0, 0)),
            pl.BlockSpec((_HID, _TPAD), lambda i: (0, 0)),
        ],
        out_specs=[
            pl.BlockSpec((_DB, _HID), lambda i: (i, 0)),
            pl.BlockSpec((8, _HID), lambda i: (0, 0)),
        ],
        out_shape=[
            jax.ShapeDtypeStruct((_B, _HID), jnp.float32),
            jax.ShapeDtypeStruct((8, _HID), jnp.float32),
        ],
    )(om, oh, dm, dh, pe, px, sv, bv, w)


def _d2_body(h_ref, v_ref, wo_ref, o_ref):
    scale = v_ref[0, :]
    shift = v_ref[1, :]
    arow = v_ref[2, :]
    y0 = h_ref[...] * scale[None, :] + shift[None, :]
    y = jnp.where(y0 >= 0.0, y0, arow[None, :] * y0)
    o_ref[...] = lax.dot_general(y, wo_ref[...], (((1,), (1,)), ((), ())),
                                 preferred_element_type=jnp.float32) + v_ref[3, 0]


def _d2(h, v, wo):
    return pl.pallas_call(
        _d2_body,
        grid=(8,),
        in_specs=[
            pl.BlockSpec((512, _HID), lambda i: (i, 0)),
            pl.BlockSpec((8, _HID), lambda i: (0, 0)),
            pl.BlockSpec((8, _HID), lambda i: (0, 0)),
        ],
        out_specs=pl.BlockSpec((512, 8), lambda i: (i, 0)),
        out_shape=jax.ShapeDtypeStruct((_B, 8), jnp.float32),
    )(h, v, wo)


def _vec8(*rows):
    n = rows[0].shape[0]
    out = [r[None, :] for r in rows]
    out.append(jnp.zeros((8 - len(rows), n), jnp.float32))
    return jnp.concatenate(out, axis=0)


def _affine(st, g, bb, bias, n):
    c = g.shape[0]
    m = st[0, :c] / n
    v = st[1, :c] / n - m * m
    scale = g * lax.rsqrt(v + 1e-5)
    shift = (bias - m) * scale + bb
    return scale, shift


def _grid_chain(x, counts, p1, p2):
    one = jnp.ones((_F,), jnp.float32)
    zero = jnp.zeros((_F,), jnp.float32)
    h1, st1, _ = _gat_pass(x, x[:_HEAD], p1['W'],
                           _vec8(one, zero, one, p1['bias']))
    hp1 = jnp.concatenate(
        [jnp.zeros((24, _F), jnp.float32), h1[:_HPR - 24]], axis=0)
    agg1, _, adj1 = _band(counts, hp1,
                          _vec8(p1['att_src'], p1['att_dst'], p1['bias']))
    sc1, sh1 = _affine(st1 + adj1, p1['bn_g'], p1['bn_b'], p1['bias'], _N)
    a1 = jnp.full((_F,), p1['a'], jnp.float32)
    h2, st2, centm = _gat_pass(h1, agg1, p2['W'],
                               _vec8(sc1, sh1, a1, p2['bias']))
    hp2 = jnp.concatenate(
        [jnp.zeros((24, _F), jnp.float32), h2[:_HPR - 24]], axis=0)
    _, centh, adj2 = _band(counts, hp2,
                           _vec8(p2['att_src'], p2['att_dst'], p2['bias']))
    sc2, sh2 = _affine(st2 + adj2, p2['bn_g'], p2['bn_b'], p2['bias'], _N)
    centh = jnp.concatenate(
        [centh, jnp.zeros((_DB - _KBLK * 8, _F), jnp.float32)], axis=0)
    return centm, centh, sc2, sh2


def kernel(o_grid_x, d_grid_x, edge_index, edge_x, external_x, params):
    counts = _sc_band_counts(edge_index.reshape(-1)).reshape(_CROWS, _CPAD)
    p1, p2 = params['gat']

    om, oh, sco, sho = _grid_chain(o_grid_x.reshape(_N, _F), counts, p1, p2)
    dm, dh, scd, shd = _grid_chain(d_grid_x.reshape(_N, _F), counts, p1, p2)

    pe_p = params['edge']
    px_p = params['ext']
    vb = _vec8(jnp.concatenate([pe_p['b'], jnp.zeros((64,), jnp.float32)]),
               jnp.concatenate([px_p['b'], jnp.zeros((96,), jnp.float32)]))
    pe, px, st0 = _d0(edge_x, external_x, pe_p['W'], px_p['W'], vb)
    sce, she = _affine(st0[0:2], pe_p['bn_g'], pe_p['bn_b'],
                       jnp.zeros((64,), jnp.float32), _B)
    scx, shx = _affine(st0[2:4], px_p['bn_g'], px_p['bn_b'],
                       jnp.zeros((32,), jnp.float32), _B)

    a2 = jnp.full((_F,), p2['a'], jnp.float32)
    z32 = jnp.zeros((32,), jnp.float32)
    scale_t = jnp.concatenate([sco, scd, sce, scx, z32])
    shift_t = jnp.concatenate([sho, shd, she, shx, z32])
    a_t = jnp.concatenate([a2, a2,
                           jnp.full((64,), pe_p['a'], jnp.float32),
                           jnp.full((32,), px_p['a'], jnp.float32),
                           jnp.ones((32,), jnp.float32)])
    p_t = params['tot1']
    w1p = jnp.concatenate(
        [p_t['W'], jnp.zeros((_HID, _TPAD - 352), jnp.float32)], axis=1)
    h, sth = _d1(om, oh, dm, dh, pe, px,
                 _vec8(scale_t, shift_t, a_t), _vec8(p_t['b']), w1p)
    sch, shh = _affine(sth, p_t['bn_g'], p_t['bn_b'],
                       jnp.zeros((_HID,), jnp.float32), _B)

    p_o = params['out']
    a_h = jnp.full((_HID,), p_t['a'], jnp.float32)
    bo_row = jnp.full((_HID,), p_o['b'][0], jnp.float32)
    wo = jnp.concatenate([p_o['W'], jnp.zeros((7, _HID), jnp.float32)], axis=0)
    out = _d2(h, _vec8(sch, shh, a_h, bo_row), wo)
    return out[:, 0]


# speedup vs baseline: 36.6368x; 1.7584x over previous
import functools

import jax
import jax.numpy as jnp
from jax import lax
from jax.experimental import pallas as pl
from jax.experimental.pallas import tpu as pltpu
from jax.experimental.pallas import tpu_sc as plsc

_B = 4096
_GRID = 25
_EPER = 48
_N = _B * _GRID
_F = 128

_RB = 200
_NBLK = _N // _RB
_KBLK = 22
_HEAD = _KBLK * _RB
_CPAD = 256
_WIN = _CPAD
_HPR = 21 * _RB + _WIN

_NW = 32
_RPW = 138
_CROWS = _NW * _RPW
_CH = 162
_EB = _CH * 2 * _EPER
_PRIV = _RPW * _CPAD


@functools.partial(
    pl.kernel,
    out_type=jax.ShapeDtypeStruct((_CROWS * _CPAD,), jnp.float32),
    mesh=plsc.VectorSubcoreMesh(core_axis_name="c", subcore_axis_name="s",
                                num_cores=2, num_subcores=16),
    scratch_types=[
        pltpu.VMEM((_EB,), jnp.int32),
        pltpu.VMEM((_PRIV,), jnp.float32),
    ],
    compiler_params=pltpu.CompilerParams(needs_layout_passes=False),
)
def _sc_band_counts(edge_ref, out_ref, edge_v, priv_v):
    cid = lax.axis_index("c")
    sid = lax.axis_index("s")
    w = sid * 2 + cid
    row0 = w * _RPW
    bstart = jnp.clip(row0 - 24, 0, _B - _CH)
    pltpu.sync_copy(edge_ref.at[pl.ds(bstart * (2 * _EPER), _EB)], edge_v)

    z16 = jnp.zeros((16,), jnp.float32)

    def _zero(i, c):
        priv_v[pl.ds(i * 16, 16)] = z16
        return c

    lax.fori_loop(0, _PRIV // 16, _zero, 0)

    lane = lax.iota(jnp.int32, 16)
    ones = jnp.ones((16,), jnp.float32)

    def _batch(bi, c):
        b = bstart + bi
        base = bi * (2 * _EPER)
        for j in range(_EPER // 16):
            sv = edge_v[pl.ds(base + j * 16, 16)]
            dv = edge_v[pl.ds(base + _EPER + j * 16, 16)]
            n = dv + b
            lrow = n - row0
            valid = (lrow >= 0) & (lrow < _RPW)
            blk = lax.div(n, _RB)
            jcol = (n - blk * _RB) + (sv - dv) + 24
            flat = jnp.where(valid, lrow * _CPAD + jcol, 0)
            for t in range(16):
                plsc.addupdate_scatter(priv_v, [flat], ones,
                                       mask=valid & (lane == t))
        return c

    lax.fori_loop(0, _CH, _batch, 0)
    pltpu.sync_copy(priv_v, out_ref.at[pl.ds(w * _PRIV, _PRIV)])


def _pass_body(xm_ref, xh_ref, w_ref, vec_ref, h_ref, st_ref, cent_ref):
    i = pl.program_id(0)
    x = jnp.where(i < _KBLK, xh_ref[...], xm_ref[...])
    scale = vec_ref[0, :]
    shift = vec_ref[1, :]
    arow = vec_ref[2, :]
    bias = vec_ref[3, :]
    y0 = x * scale[None, :] + shift[None, :]
    y = jnp.where(y0 >= 0.0, y0, arow[None, :] * y0)
    h = lax.dot_general(y, w_ref[...], (((1,), (1,)), ((), ())),
                        preferred_element_type=jnp.float32)
    h_ref[...] = h
    t = h + bias[None, :]

    @pl.when(i == 0)
    def _():
        st_ref[...] = jnp.zeros((8, _F), jnp.float32)

    st_ref[0:1, :] += jnp.sum(t, axis=0, keepdims=True)
    st_ref[1:2, :] += jnp.sum(t * t, axis=0, keepdims=True)
    cent_ref[...] = jnp.concatenate(
        [h[12 + 25 * q: 13 + 25 * q, :] for q in range(8)], axis=0)


def _gat_pass(xm, xh, w, vec):
    return pl.pallas_call(
        _pass_body,
        grid=(_NBLK,),
        in_specs=[
            pl.BlockSpec((_RB, _F), lambda i: (i, 0)),
            pl.BlockSpec((_RB, _F), lambda i: (jnp.minimum(i, _KBLK - 1), 0)),
            pl.BlockSpec((_F, _F), lambda i: (0, 0)),
            pl.BlockSpec((8, _F), lambda i: (0, 0)),
        ],
        out_specs=[
            pl.BlockSpec((_RB, _F), lambda i: (i, 0)),
            pl.BlockSpec((8, _F), lambda i: (0, 0)),
            pl.BlockSpec((8, _F), lambda i: (i, 0)),
        ],
        out_shape=[
            jax.ShapeDtypeStruct((_N, _F), jnp.float32),
            jax.ShapeDtypeStruct((8, _F), jnp.float32),
            jax.ShapeDtypeStruct((_B, _F), jnp.float32),
        ],
    )(xm, xh, w, vec)


def _band_body(cnt_ref, hp_ref, vec_ref, agg_ref, cent_ref, adj_ref):
    k = pl.program_id(0)
    hwin = hp_ref[pl.ds(k * _RB, _WIN), :]
    att_s = vec_ref[0:1, :]
    att_d = vec_ref[1:2, :]
    bias = vec_ref[2, :]
    asw = lax.dot_general(att_s, hwin, (((1,), (1,)), ((), ())),
                          preferred_element_type=jnp.float32)
    hblk = hwin[24:24 + _RB, :]
    ad = lax.dot_general(hblk, att_d, (((1,), (1,)), ((), ())),
                         preferred_element_type=jnp.float32)
    al = asw + ad
    al = jnp.where(al >= 0.0, al, 0.2 * al)
    cnt = cnt_ref[...]
    ri = lax.broadcasted_iota(jnp.int32, (_RB, _WIN), 0)
    ci = lax.broadcasted_iota(jnp.int32, (_RB, _WIN), 1)
    cnt = cnt + jnp.where(ci == ri + 24, 1.0, 0.0)
    pos = cnt > 0.0
    amax = jnp.max(jnp.where(pos, al, -1e30), axis=1, keepdims=True)
    e = cnt * jnp.exp(al - amax)
    den = jnp.sum(e, axis=1, keepdims=True)
    agg = lax.dot_general(e, hwin, (((1,), (0,)), ((), ())),
                          preferred_element_type=jnp.float32)
    agg = agg * (1.0 / (den + 1e-16))
    agg_ref[...] = agg
    cent_ref[...] = jnp.concatenate(
        [agg[12 + 25 * q: 13 + 25 * q, :] for q in range(8)], axis=0)
    ta = agg + bias[None, :]
    th = hblk + bias[None, :]

    @pl.when(k == 0)
    def _():
        adj_ref[...] = jnp.zeros((8, _F), jnp.float32)

    adj_ref[0:1, :] += (jnp.sum(ta, axis=0, keepdims=True)
                        - jnp.sum(th, axis=0, keepdims=True))
    adj_ref[1:2, :] += (jnp.sum(ta * ta, axis=0, keepdims=True)
                        - jnp.sum(th * th, axis=0, keepdims=True))


def _band(counts, hpad, vec):
    return pl.pallas_call(
        _band_body,
        grid=(_KBLK,),
        in_specs=[
            pl.BlockSpec((_RB, _CPAD), lambda k: (k, 0)),
            pl.BlockSpec((_HPR, _F), lambda k: (0, 0)),
            pl.BlockSpec((8, _F), lambda k: (0, 0)),
        ],
        out_specs=[
            pl.BlockSpec((_RB, _F), lambda k: (k, 0)),
            pl.BlockSpec((8, _F), lambda k: (k, 0)),
            pl.BlockSpec((8, _F), lambda k: (0, 0)),
        ],
        out_shape=[
            jax.ShapeDtypeStruct((_HEAD, _F), jnp.float32),
            jax.ShapeDtypeStruct((_KBLK * 8, _F), jnp.float32),
            jax.ShapeDtypeStruct((8, _F), jnp.float32),
        ],
    )(counts, hpad, vec)


def _d0_body(ex_ref, xx_ref, we_ref, wx_ref, vb_ref, pe_ref, px_ref, st_ref):
    pe = lax.dot_general(ex_ref[...], we_ref[...], (((1,), (1,)), ((), ())),
                         preferred_element_type=jnp.float32)
    pe = pe + vb_ref[0:1, :64]
    px = lax.dot_general(xx_ref[...], wx_ref[...], (((1,), (1,)), ((), ())),
                         preferred_element_type=jnp.float32)
    px = px + vb_ref[1:2, :32]
    pe_ref[...] = pe
    px_ref[...] = px
    z64 = jnp.zeros((1, 64), jnp.float32)
    z96 = jnp.zeros((1, 96), jnp.float32)
    st_ref[...] = jnp.concatenate([
        jnp.concatenate([jnp.sum(pe, 0, keepdims=True), z64], axis=1),
        jnp.concatenate([jnp.sum(pe * pe, 0, keepdims=True), z64], axis=1),
        jnp.concatenate([jnp.sum(px, 0, keepdims=True), z96], axis=1),
        jnp.concatenate([jnp.sum(px * px, 0, keepdims=True), z96], axis=1),
        jnp.zeros((4, _F), jnp.float32),
    ], axis=0)


def _d0(edge_x, external_x, we, wx, vb):
    return pl.pallas_call(
        _d0_body,
        grid=(1,),
        in_specs=[
            pl.BlockSpec((_B, 32), lambda i: (0, 0)),
            pl.BlockSpec((_B, 16), lambda i: (0, 0)),
            pl.BlockSpec((64, 32), lambda i: (0, 0)),
            pl.BlockSpec((32, 16), lambda i: (0, 0)),
            pl.BlockSpec((8, _F), lambda i: (0, 0)),
        ],
        out_specs=[
            pl.BlockSpec((_B, 64), lambda i: (0, 0)),
            pl.BlockSpec((_B, 32), lambda i: (0, 0)),
            pl.BlockSpec((8, _F), lambda i: (0, 0)),
        ],
        out_shape=[
            jax.ShapeDtypeStruct((_B, 64), jnp.float32),
            jax.ShapeDtypeStruct((_B, 32), jnp.float32),
            jax.ShapeDtypeStruct((8, _F), jnp.float32),
        ],
    )(edge_x, external_x, we, wx, vb)


_DB = 256
_TPAD = 384
_HID = 256


def _d1_body(om_ref, oh_ref, dm_ref, dh_ref, pe_ref, px_ref, sv_ref, bv_ref,
             w_ref, h_ref, st_ref):
    i = pl.program_id(0)
    ri = lax.broadcasted_iota(jnp.int32, (_DB, 1), 0) + i * _DB
    use_h = ri < 176
    o_raw = jnp.where(use_h, oh_ref[...], om_ref[...])
    d_raw = jnp.where(use_h, dh_ref[...], dm_ref[...])
    tot = jnp.concatenate(
        [o_raw, d_raw, pe_ref[...], px_ref[...],
         jnp.zeros((_DB, 32), jnp.float32)], axis=1)
    scale = sv_ref[0, :]
    shift = sv_ref[1, :]
    arow = sv_ref[2, :]
    y0 = tot * scale[None, :] + shift[None, :]
    y = jnp.where(y0 >= 0.0, y0, arow[None, :] * y0)
    h = lax.dot_general(y, w_ref[...], (((1,), (1,)), ((), ())),
                        preferred_element_type=jnp.float32)
    h = h + bv_ref[0:1, :]
    h_ref[...] = h

    @pl.when(i == 0)
    def _():
        st_ref[...] = jnp.zeros((8, _HID), jnp.float32)

    st_ref[0:1, :] += jnp.sum(h, axis=0, keepdims=True)
    st_ref[1:2, :] += jnp.sum(h * h, axis=0, keepdims=True)


def _d1(om, oh, dm, dh, pe, px, sv, bv, w):
    return pl.pallas_call(
        _d1_body,
        grid=(_B // _DB,),
        in_specs=[
            pl.BlockSpec((_DB, _F), lambda i: (i, 0)),
            pl.BlockSpec((_DB, _F), lambda i: (0, 0)),
            pl.BlockSpec((_DB, _F), lambda i: (i, 0)),
            pl.BlockSpec((_DB, _F), lambda i: (0, 0)),
            pl.BlockSpec((_DB, 64), lambda i: (i, 0)),
            pl.BlockSpec((_DB, 32), lambda i: (i, 0)),
            pl.BlockSpec((8, _TPAD), lambda i: (0, 0)),
            pl.BlockSpec((8, _HID), lambda i: (0, 0)),
            pl.BlockSpec((_HID, _TPAD), lambda i: (0, 0)),
        ],
        out_specs=[
            pl.BlockSpec((_DB, _HID), lambda i: (i, 0)),
            pl.BlockSpec((8, _HID), lambda i: (0, 0)),
        ],
        out_shape=[
            jax.ShapeDtypeStruct((_B, _HID), jnp.float32),
            jax.ShapeDtypeStruct((8, _HID), jnp.float32),
        ],
    )(om, oh, dm, dh, pe, px, sv, bv, w)


def _d2_body(h_ref, v_ref, wo_ref, o_ref):
    scale = v_ref[0, :]
    shift = v_ref[1, :]
    arow = v_ref[2, :]
    y0 = h_ref[...] * scale[None, :] + shift[None, :]
    y = jnp.where(y0 >= 0.0, y0, arow[None, :] * y0)
    o_ref[...] = lax.dot_general(y, wo_ref[...], (((1,), (1,)), ((), ())),
                                 preferred_element_type=jnp.float32) + v_ref[3, 0]


def _d2(h, v, wo):
    return pl.pallas_call(
        _d2_body,
        grid=(8,),
        in_specs=[
            pl.BlockSpec((512, _HID), lambda i: (i, 0)),
            pl.BlockSpec((8, _HID), lambda i: (0, 0)),
            pl.BlockSpec((8, _HID), lambda i: (0, 0)),
        ],
        out_specs=pl.BlockSpec((512, 8), lambda i: (i, 0)),
        out_shape=jax.ShapeDtypeStruct((_B, 8), jnp.float32),
    )(h, v, wo)


def _vec8(*rows):
    n = rows[0].shape[0]
    out = [r[None, :] for r in rows]
    out.append(jnp.zeros((8 - len(rows), n), jnp.float32))
    return jnp.concatenate(out, axis=0)


def _affine(st, g, bb, bias, n):
    c = g.shape[0]
    m = st[0, :c] / n
    v = st[1, :c] / n - m * m
    scale = g * lax.rsqrt(v + 1e-5)
    shift = (bias - m) * scale + bb
    return scale, shift


def _grid_chain(x, counts, p1, p2):
    one = jnp.ones((_F,), jnp.float32)
    zero = jnp.zeros((_F,), jnp.float32)
    h1, st1, _ = _gat_pass(x, x[:_HEAD], p1['W'],
                           _vec8(one, zero, one, p1['bias']))
    hp1 = jnp.concatenate(
        [jnp.zeros((24, _F), jnp.float32), h1[:_HPR - 24]], axis=0)
    agg1, _, adj1 = _band(counts, hp1,
                          _vec8(p1['att_src'], p1['att_dst'], p1['bias']))
    sc1, sh1 = _affine(st1 + adj1, p1['bn_g'], p1['bn_b'], p1['bias'], _N)
    a1 = jnp.full((_F,), p1['a'], jnp.float32)
    h2, st2, centm = _gat_pass(h1, agg1, p2['W'],
                               _vec8(sc1, sh1, a1, p2['bias']))
    hp2 = jnp.concatenate(
        [jnp.zeros((24, _F), jnp.float32), h2[:_HPR - 24]], axis=0)
    _, centh, adj2 = _band(counts, hp2,
                           _vec8(p2['att_src'], p2['att_dst'], p2['bias']))
    sc2, sh2 = _affine(st2 + adj2, p2['bn_g'], p2['bn_b'], p2['bias'], _N)
    centh = jnp.concatenate(
        [centh, jnp.zeros((_DB - _KBLK * 8, _F), jnp.float32)], axis=0)
    return centm, centh, sc2, sh2


def kernel(o_grid_x, d_grid_x, edge_index, edge_x, external_x, params):
    counts = _sc_band_counts(edge_index.reshape(-1)).reshape(_CROWS, _CPAD)
    p1, p2 = params['gat']

    om, oh, sco, sho = _grid_chain(o_grid_x.reshape(_N, _F), counts, p1, p2)
    dm, dh, scd, shd = _grid_chain(d_grid_x.reshape(_N, _F), counts, p1, p2)

    pe_p = params['edge']
    px_p = params['ext']
    vb = _vec8(jnp.concatenate([pe_p['b'], jnp.zeros((64,), jnp.float32)]),
               jnp.concatenate([px_p['b'], jnp.zeros((96,), jnp.float32)]))
    pe, px, st0 = _d0(edge_x, external_x, pe_p['W'], px_p['W'], vb)
    sce, she = _affine(st0[0:2], pe_p['bn_g'], pe_p['bn_b'],
                       jnp.zeros((64,), jnp.float32), _B)
    scx, shx = _affine(st0[2:4], px_p['bn_g'], px_p['bn_b'],
                       jnp.zeros((32,), jnp.float32), _B)

    a2 = jnp.full((_F,), p2['a'], jnp.float32)
    z32 = jnp.zeros((32,), jnp.float32)
    scale_t = jnp.concatenate([sco, scd, sce, scx, z32])
    shift_t = jnp.concatenate([sho, shd, she, shx, z32])
    a_t = jnp.concatenate([a2, a2,
                           jnp.full((64,), pe_p['a'], jnp.float32),
                           jnp.full((32,), px_p['a'], jnp.float32),
                           jnp.ones((32,), jnp.float32)])
    p_t = params['tot1']
    w1p = jnp.concatenate(
        [p_t['W'], jnp.zeros((_HID, _TPAD - 352), jnp.float32)], axis=1)
    h, sth = _d1(om, oh, dm, dh, pe, px,
                 _vec8(scale_t, shift_t, a_t), _vec8(p_t['b']), w1p)
    sch, shh = _affine(sth, p_t['bn_g'], p_t['bn_b'],
                       jnp.zeros((_HID,), jnp.float32), _B)

    p_o = params['out']
    a_h = jnp.full((_HID,), p_t['a'], jnp.float32)
    bo_row = jnp.full((_HID,), p_o['b'][0], jnp.float32)
    wo = jnp.concatenate([p_o['W'], jnp.zeros((7, _HID), jnp.float32)], axis=0)
    out = _d2(h, _vec8(sch, shh, a_h, bo_row), wo)
    return out[:, 0]


# speedup vs baseline: 75.8664x; 2.0708x over previous
import functools

import jax
import jax.numpy as jnp
from jax import lax
from jax.experimental import pallas as pl
from jax.experimental.pallas import tpu as pltpu
from jax.experimental.pallas import tpu_sc as plsc

_B = 4096
_GRID = 25
_EPER = 48
_N = _B * _GRID
_F = 128

_RB = 800
_NBLK = _N // _RB
_RBB = 200
_KBLK = 24
_HEAD = _KBLK * _RBB
_KH = _HEAD // _RB
_CPAD = 256
_WIN = _CPAD
_HPR = (_KBLK - 1) * _RBB + _WIN

_NW = 32
_RPW = 150
_CROWS = _NW * _RPW
_CH = 174
_EB = _CH * 2 * _EPER
_PRIV = _RPW * _CPAD


@functools.partial(
    pl.kernel,
    out_type=jax.ShapeDtypeStruct((_CROWS * _CPAD,), jnp.float32),
    mesh=plsc.VectorSubcoreMesh(core_axis_name="c", subcore_axis_name="s",
                                num_cores=2, num_subcores=16),
    scratch_types=[
        pltpu.VMEM((_EB,), jnp.int32),
        pltpu.VMEM((_PRIV,), jnp.float32),
    ],
    compiler_params=pltpu.CompilerParams(needs_layout_passes=False),
)
def _sc_band_counts(edge_ref, out_ref, edge_v, priv_v):
    cid = lax.axis_index("c")
    sid = lax.axis_index("s")
    w = sid * 2 + cid
    row0 = w * _RPW
    bstart = jnp.clip(row0 - 24, 0, _B - _CH)
    pltpu.sync_copy(edge_ref.at[pl.ds(bstart * (2 * _EPER), _EB)], edge_v)

    z16 = jnp.zeros((16,), jnp.float32)

    def _zero(i, c):
        priv_v[pl.ds(i * 16, 16)] = z16
        return c

    lax.fori_loop(0, _PRIV // 16, _zero, 0)

    lane = lax.iota(jnp.int32, 16)
    ones = jnp.ones((16,), jnp.float32)

    def _batch(bi, c):
        b = bstart + bi
        base = bi * (2 * _EPER)
        for j in range(_EPER // 16):
            sv = edge_v[pl.ds(base + j * 16, 16)]
            dv = edge_v[pl.ds(base + _EPER + j * 16, 16)]
            n = dv + b
            lrow = n - row0
            valid = (lrow >= 0) & (lrow < _RPW)
            blk = lax.div(n, _RBB)
            jcol = (n - blk * _RBB) + (sv - dv) + 24
            flat = jnp.where(valid, lrow * _CPAD + jcol, 0)
            for t in range(16):
                plsc.addupdate_scatter(priv_v, [flat], ones,
                                       mask=valid & (lane == t))
        return c

    lax.fori_loop(0, _CH, _batch, 0)
    pltpu.sync_copy(priv_v, out_ref.at[pl.ds(w * _PRIV, _PRIV)])


def _pass_body(xm_ref, xh_ref, w_ref, vec_ref, h_ref, st_ref, cent_ref):
    i = pl.program_id(0)
    x = jnp.where(i < _KH, xh_ref[...], xm_ref[...])
    scale = vec_ref[0, :]
    shift = vec_ref[1, :]
    arow = vec_ref[2, :]
    bias = vec_ref[3, :]
    y0 = x * scale[None, :] + shift[None, :]
    y = jnp.where(y0 >= 0.0, y0, arow[None, :] * y0)
    h = lax.dot_general(y, w_ref[...], (((1,), (1,)), ((), ())),
                        preferred_element_type=jnp.float32)
    h_ref[...] = h
    t = h + bias[None, :]

    @pl.when(i == 0)
    def _():
        st_ref[...] = jnp.zeros((8, _F), jnp.float32)

    st_ref[0:1, :] += jnp.sum(t, axis=0, keepdims=True)
    st_ref[1:2, :] += jnp.sum(t * t, axis=0, keepdims=True)
    cent_ref[...] = jnp.concatenate(
        [h[12 + 25 * q: 13 + 25 * q, :] for q in range(_RB // 25)], axis=0)


def _gat_pass(xm, xh, w, vec):
    return pl.pallas_call(
        _pass_body,
        grid=(_NBLK,),
        in_specs=[
            pl.BlockSpec((_RB, _F), lambda i: (i, 0)),
            pl.BlockSpec((_RB, _F), lambda i: (jnp.minimum(i, _KH - 1), 0)),
            pl.BlockSpec((_F, _F), lambda i: (0, 0)),
            pl.BlockSpec((8, _F), lambda i: (0, 0)),
        ],
        out_specs=[
            pl.BlockSpec((_RB, _F), lambda i: (i, 0)),
            pl.BlockSpec((8, _F), lambda i: (0, 0)),
            pl.BlockSpec((_RB // 25, _F), lambda i: (i, 0)),
        ],
        out_shape=[
            jax.ShapeDtypeStruct((_N, _F), jnp.float32),
            jax.ShapeDtypeStruct((8, _F), jnp.float32),
            jax.ShapeDtypeStruct((_B, _F), jnp.float32),
        ],
    )(xm, xh, w, vec)


def _band_body(cnt_ref, hp_ref, vec_ref, agg_ref, cent_ref, adj_ref):
    k = pl.program_id(0)
    hwin = hp_ref[pl.ds(k * _RBB, _WIN), :]
    att_s = vec_ref[0:1, :]
    att_d = vec_ref[1:2, :]
    bias = vec_ref[2, :]
    asw = lax.dot_general(att_s, hwin, (((1,), (1,)), ((), ())),
                          preferred_element_type=jnp.float32)
    hblk = hwin[24:24 + _RBB, :]
    ad = lax.dot_general(hblk, att_d, (((1,), (1,)), ((), ())),
                         preferred_element_type=jnp.float32)
    al = asw + ad
    al = jnp.where(al >= 0.0, al, 0.2 * al)
    cnt = cnt_ref[...]
    ri = lax.broadcasted_iota(jnp.int32, (_RBB, _WIN), 0)
    ci = lax.broadcasted_iota(jnp.int32, (_RBB, _WIN), 1)
    cnt = cnt + jnp.where(ci == ri + 24, 1.0, 0.0)
    pos = cnt > 0.0
    amax = jnp.max(jnp.where(pos, al, -1e30), axis=1, keepdims=True)
    e = cnt * jnp.exp(al - amax)
    den = jnp.sum(e, axis=1, keepdims=True)
    agg = lax.dot_general(e, hwin, (((1,), (0,)), ((), ())),
                          preferred_element_type=jnp.float32)
    agg = agg * (1.0 / (den + 1e-16))
    agg_ref[...] = agg
    cent_ref[...] = jnp.concatenate(
        [agg[12 + 25 * q: 13 + 25 * q, :] for q in range(8)], axis=0)
    ta = agg + bias[None, :]
    th = hblk + bias[None, :]

    @pl.when(k == 0)
    def _():
        adj_ref[...] = jnp.zeros((8, _F), jnp.float32)

    adj_ref[0:1, :] += (jnp.sum(ta, axis=0, keepdims=True)
                        - jnp.sum(th, axis=0, keepdims=True))
    adj_ref[1:2, :] += (jnp.sum(ta * ta, axis=0, keepdims=True)
                        - jnp.sum(th * th, axis=0, keepdims=True))


def _band(counts, hpad, vec):
    return pl.pallas_call(
        _band_body,
        grid=(_KBLK,),
        in_specs=[
            pl.BlockSpec((_RBB, _CPAD), lambda k: (k, 0)),
            pl.BlockSpec((_HPR, _F), lambda k: (0, 0)),
            pl.BlockSpec((8, _F), lambda k: (0, 0)),
        ],
        out_specs=[
            pl.BlockSpec((_RBB, _F), lambda k: (k, 0)),
            pl.BlockSpec((8, _F), lambda k: (k, 0)),
            pl.BlockSpec((8, _F), lambda k: (0, 0)),
        ],
        out_shape=[
            jax.ShapeDtypeStruct((_HEAD, _F), jnp.float32),
            jax.ShapeDtypeStruct((_KBLK * 8, _F), jnp.float32),
            jax.ShapeDtypeStruct((8, _F), jnp.float32),
        ],
    )(counts, hpad, vec)


def _d0_body(ex_ref, xx_ref, we_ref, wx_ref, vb_ref, pe_ref, px_ref, st_ref):
    pe = lax.dot_general(ex_ref[...], we_ref[...], (((1,), (1,)), ((), ())),
                         preferred_element_type=jnp.float32)
    pe = pe + vb_ref[0:1, :64]
    px = lax.dot_general(xx_ref[...], wx_ref[...], (((1,), (1,)), ((), ())),
                         preferred_element_type=jnp.float32)
    px = px + vb_ref[1:2, :32]
    pe_ref[...] = pe
    px_ref[...] = px
    z64 = jnp.zeros((1, 64), jnp.float32)
    z96 = jnp.zeros((1, 96), jnp.float32)
    st_ref[...] = jnp.concatenate([
        jnp.concatenate([jnp.sum(pe, 0, keepdims=True), z64], axis=1),
        jnp.concatenate([jnp.sum(pe * pe, 0, keepdims=True), z64], axis=1),
        jnp.concatenate([jnp.sum(px, 0, keepdims=True), z96], axis=1),
        jnp.concatenate([jnp.sum(px * px, 0, keepdims=True), z96], axis=1),
        jnp.zeros((4, _F), jnp.float32),
    ], axis=0)


def _d0(edge_x, external_x, we, wx, vb):
    return pl.pallas_call(
        _d0_body,
        grid=(1,),
        in_specs=[
            pl.BlockSpec((_B, 32), lambda i: (0, 0)),
            pl.BlockSpec((_B, 16), lambda i: (0, 0)),
            pl.BlockSpec((64, 32), lambda i: (0, 0)),
            pl.BlockSpec((32, 16), lambda i: (0, 0)),
            pl.BlockSpec((8, _F), lambda i: (0, 0)),
        ],
        out_specs=[
            pl.BlockSpec((_B, 64), lambda i: (0, 0)),
            pl.BlockSpec((_B, 32), lambda i: (0, 0)),
            pl.BlockSpec((8, _F), lambda i: (0, 0)),
        ],
        out_shape=[
            jax.ShapeDtypeStruct((_B, 64), jnp.float32),
            jax.ShapeDtypeStruct((_B, 32), jnp.float32),
            jax.ShapeDtypeStruct((8, _F), jnp.float32),
        ],
    )(edge_x, external_x, we, wx, vb)


_DB = 256
_TPAD = 384
_HID = 256


def _d1_body(om_ref, oh_ref, dm_ref, dh_ref, pe_ref, px_ref, sv_ref, bv_ref,
             w_ref, h_ref, st_ref):
    i = pl.program_id(0)
    ri = lax.broadcasted_iota(jnp.int32, (_DB, 1), 0) + i * _DB
    use_h = ri < 176
    o_raw = jnp.where(use_h, oh_ref[...], om_ref[...])
    d_raw = jnp.where(use_h, dh_ref[...], dm_ref[...])
    tot = jnp.concatenate(
        [o_raw, d_raw, pe_ref[...], px_ref[...],
         jnp.zeros((_DB, 32), jnp.float32)], axis=1)
    scale = sv_ref[0, :]
    shift = sv_ref[1, :]
    arow = sv_ref[2, :]
    y0 = tot * scale[None, :] + shift[None, :]
    y = jnp.where(y0 >= 0.0, y0, arow[None, :] * y0)
    h = lax.dot_general(y, w_ref[...], (((1,), (1,)), ((), ())),
                        preferred_element_type=jnp.float32)
    h = h + bv_ref[0:1, :]
    h_ref[...] = h

    @pl.when(i == 0)
    def _():
        st_ref[...] = jnp.zeros((8, _HID), jnp.float32)

    st_ref[0:1, :] += jnp.sum(h, axis=0, keepdims=True)
    st_ref[1:2, :] += jnp.sum(h * h, axis=0, keepdims=True)


def _d1(om, oh, dm, dh, pe, px, sv, bv, w):
    return pl.pallas_call(
        _d1_body,
        grid=(_B // _DB,),
        in_specs=[
            pl.BlockSpec((_DB, _F), lambda i: (i, 0)),
            pl.BlockSpec((_DB, _F), lambda i: (0, 0)),
            pl.BlockSpec((_DB, _F), lambda i: (i, 0)),
            pl.BlockSpec((_DB, _F), lambda i: (0, 0)),
            pl.BlockSpec((_DB, 64), lambda i: (i, 0)),
            pl.BlockSpec((_DB, 32), lambda i: (i, 0)),
            pl.BlockSpec((8, _TPAD), lambda i: (0, 0)),
            pl.BlockSpec((8, _HID), lambda i: (0, 0)),
            pl.BlockSpec((_HID, _TPAD), lambda i: (0, 0)),
        ],
        out_specs=[
            pl.BlockSpec((_DB, _HID), lambda i: (i, 0)),
            pl.BlockSpec((8, _HID), lambda i: (0, 0)),
        ],
        out_shape=[
            jax.ShapeDtypeStruct((_B, _HID), jnp.float32),
            jax.ShapeDtypeStruct((8, _HID), jnp.float32),
        ],
    )(om, oh, dm, dh, pe, px, sv, bv, w)


def _d2_body(h_ref, v_ref, wo_ref, o_ref):
    scale = v_ref[0, :]
    shift = v_ref[1, :]
    arow = v_ref[2, :]
    y0 = h_ref[...] * scale[None, :] + shift[None, :]
    y = jnp.where(y0 >= 0.0, y0, arow[None, :] * y0)
    o_ref[...] = lax.dot_general(y, wo_ref[...], (((1,), (1,)), ((), ())),
                                 preferred_element_type=jnp.float32) + v_ref[3, 0]


def _d2(h, v, wo):
    return pl.pallas_call(
        _d2_body,
        grid=(8,),
        in_specs=[
            pl.BlockSpec((512, _HID), lambda i: (i, 0)),
            pl.BlockSpec((8, _HID), lambda i: (0, 0)),
            pl.BlockSpec((8, _HID), lambda i: (0, 0)),
        ],
        out_specs=pl.BlockSpec((512, 8), lambda i: (i, 0)),
        out_shape=jax.ShapeDtypeStruct((_B, 8), jnp.float32),
    )(h, v, wo)


def _vec8(*rows):
    n = rows[0].shape[0]
    out = [r[None, :] for r in rows]
    out.append(jnp.zeros((8 - len(rows), n), jnp.float32))
    return jnp.concatenate(out, axis=0)


def _affine(st, g, bb, bias, n):
    c = g.shape[0]
    m = st[0, :c] / n
    v = st[1, :c] / n - m * m
    scale = g * lax.rsqrt(v + 1e-5)
    shift = (bias - m) * scale + bb
    return scale, shift


def _grid_chain(x, counts, p1, p2):
    one = jnp.ones((_F,), jnp.float32)
    zero = jnp.zeros((_F,), jnp.float32)
    h1, st1, _ = _gat_pass(x, x[:_HEAD], p1['W'],
                           _vec8(one, zero, one, p1['bias']))
    hp1 = jnp.concatenate(
        [jnp.zeros((24, _F), jnp.float32), h1[:_HPR - 24]], axis=0)
    agg1, _, adj1 = _band(counts, hp1,
                          _vec8(p1['att_src'], p1['att_dst'], p1['bias']))
    sc1, sh1 = _affine(st1 + adj1, p1['bn_g'], p1['bn_b'], p1['bias'], _N)
    a1 = jnp.full((_F,), p1['a'], jnp.float32)
    h2, st2, centm = _gat_pass(h1, agg1, p2['W'],
                               _vec8(sc1, sh1, a1, p2['bias']))
    hp2 = jnp.concatenate(
        [jnp.zeros((24, _F), jnp.float32), h2[:_HPR - 24]], axis=0)
    _, centh, adj2 = _band(counts, hp2,
                           _vec8(p2['att_src'], p2['att_dst'], p2['bias']))
    sc2, sh2 = _affine(st2 + adj2, p2['bn_g'], p2['bn_b'], p2['bias'], _N)
    centh = jnp.concatenate(
        [centh, jnp.zeros((_DB - _KBLK * 8, _F), jnp.float32)], axis=0)
    return centm, centh, sc2, sh2


def kernel(o_grid_x, d_grid_x, edge_index, edge_x, external_x, params):
    counts = _sc_band_counts(edge_index.reshape(-1)).reshape(_CROWS, _CPAD)
    p1, p2 = params['gat']

    om, oh, sco, sho = _grid_chain(o_grid_x.reshape(_N, _F), counts, p1, p2)
    dm, dh, scd, shd = _grid_chain(d_grid_x.reshape(_N, _F), counts, p1, p2)

    pe_p = params['edge']
    px_p = params['ext']
    vb = _vec8(jnp.concatenate([pe_p['b'], jnp.zeros((64,), jnp.float32)]),
               jnp.concatenate([px_p['b'], jnp.zeros((96,), jnp.float32)]))
    pe, px, st0 = _d0(edge_x, external_x, pe_p['W'], px_p['W'], vb)
    sce, she = _affine(st0[0:2], pe_p['bn_g'], pe_p['bn_b'],
                       jnp.zeros((64,), jnp.float32), _B)
    scx, shx = _affine(st0[2:4], px_p['bn_g'], px_p['bn_b'],
                       jnp.zeros((32,), jnp.float32), _B)

    a2 = jnp.full((_F,), p2['a'], jnp.float32)
    z32 = jnp.zeros((32,), jnp.float32)
    scale_t = jnp.concatenate([sco, scd, sce, scx, z32])
    shift_t = jnp.concatenate([sho, shd, she, shx, z32])
    a_t = jnp.concatenate([a2, a2,
                           jnp.full((64,), pe_p['a'], jnp.float32),
                           jnp.full((32,), px_p['a'], jnp.float32),
                           jnp.ones((32,), jnp.float32)])
    p_t = params['tot1']
    w1p = jnp.concatenate(
        [p_t['W'], jnp.zeros((_HID, _TPAD - 352), jnp.float32)], axis=1)
    h, sth = _d1(om, oh, dm, dh, pe, px,
                 _vec8(scale_t, shift_t, a_t), _vec8(p_t['b']), w1p)
    sch, shh = _affine(sth, p_t['bn_g'], p_t['bn_b'],
                       jnp.zeros((_HID,), jnp.float32), _B)

    p_o = params['out']
    a_h = jnp.full((_HID,), p_t['a'], jnp.float32)
    bo_row = jnp.full((_HID,), p_o['b'][0], jnp.float32)
    wo = jnp.concatenate([p_o['W'], jnp.zeros((7, _HID), jnp.float32)], axis=0)
    out = _d2(h, _vec8(sch, shh, a_h, bo_row), wo)
    return out[:, 0]


# speedup vs baseline: 90.5813x; 1.1940x over previous
import functools

import jax
import jax.numpy as jnp
from jax import lax
from jax.experimental import pallas as pl
from jax.experimental.pallas import tpu as pltpu
from jax.experimental.pallas import tpu_sc as plsc

_B = 4096
_GRID = 25
_EPER = 48
_N = _B * _GRID
_F = 128

_RB = 1600
_NBLK = _N // _RB
_RBB = 200
_KBLK = 24
_HEAD = _KBLK * _RBB
_KH = _HEAD // _RB
_CPAD = 256
_WIN = _CPAD
_HPR = (_KBLK - 1) * _RBB + _WIN

_NW = 32
_RPW = 150
_CROWS = _NW * _RPW
_CH = 174
_EB = _CH * 2 * _EPER
_PRIV = _RPW * _CPAD


@functools.partial(
    pl.kernel,
    out_type=jax.ShapeDtypeStruct((_CROWS * _CPAD,), jnp.float32),
    mesh=plsc.VectorSubcoreMesh(core_axis_name="c", subcore_axis_name="s",
                                num_cores=2, num_subcores=16),
    scratch_types=[
        pltpu.VMEM((_EB,), jnp.int32),
        pltpu.VMEM((_PRIV,), jnp.float32),
    ],
    compiler_params=pltpu.CompilerParams(needs_layout_passes=False),
)
def _sc_band_counts(edge_ref, out_ref, edge_v, priv_v):
    cid = lax.axis_index("c")
    sid = lax.axis_index("s")
    w = sid * 2 + cid
    row0 = w * _RPW
    bstart = jnp.clip(row0 - 24, 0, _B - _CH)
    pltpu.sync_copy(edge_ref.at[pl.ds(bstart * (2 * _EPER), _EB)], edge_v)

    z16 = jnp.zeros((16,), jnp.float32)

    def _zero(i, c):
        priv_v[pl.ds(i * 16, 16)] = z16
        return c

    lax.fori_loop(0, _PRIV // 16, _zero, 0)

    lane = lax.iota(jnp.int32, 16)
    ones = jnp.ones((16,), jnp.float32)

    def _batch(bi, c):
        b = bstart + bi
        base = bi * (2 * _EPER)
        for j in range(_EPER // 16):
            sv = edge_v[pl.ds(base + j * 16, 16)]
            dv = edge_v[pl.ds(base + _EPER + j * 16, 16)]
            n = dv + b
            lrow = n - row0
            valid = (lrow >= 0) & (lrow < _RPW)
            blk = lax.div(n, _RBB)
            jcol = (n - blk * _RBB) + (sv - dv) + 24
            flat = jnp.where(valid, lrow * _CPAD + jcol, 0)
            for t in range(16):
                plsc.addupdate_scatter(priv_v, [flat], ones,
                                       mask=valid & (lane == t))
        return c

    lax.fori_loop(0, _CH, _batch, 0)
    pltpu.sync_copy(priv_v, out_ref.at[pl.ds(w * _PRIV, _PRIV)])


def _pass_body(xm_ref, xh_ref, w_ref, vec_ref, h_ref, st_ref, cent_ref):
    i = pl.program_id(0)
    x = jnp.where(i < _KH, xh_ref[...], xm_ref[...])
    scale = vec_ref[0, :]
    shift = vec_ref[1, :]
    arow = vec_ref[2, :]
    bias = vec_ref[3, :]
    y0 = x * scale[None, :] + shift[None, :]
    y = jnp.where(y0 >= 0.0, y0, arow[None, :] * y0)
    h = lax.dot_general(y, w_ref[...], (((1,), (1,)), ((), ())),
                        preferred_element_type=jnp.float32)
    h_ref[...] = h
    t = h + bias[None, :]

    @pl.when(i == 0)
    def _():
        st_ref[...] = jnp.zeros((8, _F), jnp.float32)

    st_ref[0:1, :] += jnp.sum(t, axis=0, keepdims=True)
    st_ref[1:2, :] += jnp.sum(t * t, axis=0, keepdims=True)
    cent_ref[...] = jnp.concatenate(
        [h[12 + 25 * q: 13 + 25 * q, :] for q in range(_RB // 25)], axis=0)


def _gat_pass(xm, xh, w, vec):
    return pl.pallas_call(
        _pass_body,
        grid=(_NBLK,),
        in_specs=[
            pl.BlockSpec((_RB, _F), lambda i: (i, 0)),
            pl.BlockSpec((_RB, _F), lambda i: (jnp.minimum(i, _KH - 1), 0)),
            pl.BlockSpec((_F, _F), lambda i: (0, 0)),
            pl.BlockSpec((8, _F), lambda i: (0, 0)),
        ],
        out_specs=[
            pl.BlockSpec((_RB, _F), lambda i: (i, 0)),
            pl.BlockSpec((8, _F), lambda i: (0, 0)),
            pl.BlockSpec((_RB // 25, _F), lambda i: (i, 0)),
        ],
        out_shape=[
            jax.ShapeDtypeStruct((_N, _F), jnp.float32),
            jax.ShapeDtypeStruct((8, _F), jnp.float32),
            jax.ShapeDtypeStruct((_B, _F), jnp.float32),
        ],
    )(xm, xh, w, vec)


def _band_body(cnt_ref, hp_ref, vec_ref, agg_ref, cent_ref, adj_ref):
    k = pl.program_id(0)
    hwin = hp_ref[pl.ds(k * _RBB, _WIN), :]
    att_s = vec_ref[0:1, :]
    att_d = vec_ref[1:2, :]
    bias = vec_ref[2, :]
    asw = lax.dot_general(att_s, hwin, (((1,), (1,)), ((), ())),
                          preferred_element_type=jnp.float32)
    hblk = hwin[24:24 + _RBB, :]
    ad = lax.dot_general(hblk, att_d, (((1,), (1,)), ((), ())),
                         preferred_element_type=jnp.float32)
    al = asw + ad
    al = jnp.where(al >= 0.0, al, 0.2 * al)
    cnt = cnt_ref[...]
    ri = lax.broadcasted_iota(jnp.int32, (_RBB, _WIN), 0)
    ci = lax.broadcasted_iota(jnp.int32, (_RBB, _WIN), 1)
    cnt = cnt + jnp.where(ci == ri + 24, 1.0, 0.0)
    pos = cnt > 0.0
    amax = jnp.max(jnp.where(pos, al, -1e30), axis=1, keepdims=True)
    e = cnt * jnp.exp(al - amax)
    den = jnp.sum(e, axis=1, keepdims=True)
    agg = lax.dot_general(e, hwin, (((1,), (0,)), ((), ())),
                          preferred_element_type=jnp.float32)
    agg = agg * (1.0 / (den + 1e-16))
    agg_ref[...] = agg
    cent_ref[...] = jnp.concatenate(
        [agg[12 + 25 * q: 13 + 25 * q, :] for q in range(8)], axis=0)
    ta = agg + bias[None, :]
    th = hblk + bias[None, :]

    @pl.when(k == 0)
    def _():
        adj_ref[...] = jnp.zeros((8, _F), jnp.float32)

    adj_ref[0:1, :] += (jnp.sum(ta, axis=0, keepdims=True)
                        - jnp.sum(th, axis=0, keepdims=True))
    adj_ref[1:2, :] += (jnp.sum(ta * ta, axis=0, keepdims=True)
                        - jnp.sum(th * th, axis=0, keepdims=True))


def _band(counts, hpad, vec):
    return pl.pallas_call(
        _band_body,
        grid=(_KBLK,),
        in_specs=[
            pl.BlockSpec((_RBB, _CPAD), lambda k: (k, 0)),
            pl.BlockSpec((_HPR, _F), lambda k: (0, 0)),
            pl.BlockSpec((8, _F), lambda k: (0, 0)),
        ],
        out_specs=[
            pl.BlockSpec((_RBB, _F), lambda k: (k, 0)),
            pl.BlockSpec((8, _F), lambda k: (k, 0)),
            pl.BlockSpec((8, _F), lambda k: (0, 0)),
        ],
        out_shape=[
            jax.ShapeDtypeStruct((_HEAD, _F), jnp.float32),
            jax.ShapeDtypeStruct((_KBLK * 8, _F), jnp.float32),
            jax.ShapeDtypeStruct((8, _F), jnp.float32),
        ],
    )(counts, hpad, vec)


def _d0_body(ex_ref, xx_ref, we_ref, wx_ref, vb_ref, pe_ref, px_ref, st_ref):
    pe = lax.dot_general(ex_ref[...], we_ref[...], (((1,), (1,)), ((), ())),
                         preferred_element_type=jnp.float32)
    pe = pe + vb_ref[0:1, :64]
    px = lax.dot_general(xx_ref[...], wx_ref[...], (((1,), (1,)), ((), ())),
                         preferred_element_type=jnp.float32)
    px = px + vb_ref[1:2, :32]
    pe_ref[...] = pe
    px_ref[...] = px
    z64 = jnp.zeros((1, 64), jnp.float32)
    z96 = jnp.zeros((1, 96), jnp.float32)
    st_ref[...] = jnp.concatenate([
        jnp.concatenate([jnp.sum(pe, 0, keepdims=True), z64], axis=1),
        jnp.concatenate([jnp.sum(pe * pe, 0, keepdims=True), z64], axis=1),
        jnp.concatenate([jnp.sum(px, 0, keepdims=True), z96], axis=1),
        jnp.concatenate([jnp.sum(px * px, 0, keepdims=True), z96], axis=1),
        jnp.zeros((4, _F), jnp.float32),
    ], axis=0)


def _d0(edge_x, external_x, we, wx, vb):
    return pl.pallas_call(
        _d0_body,
        grid=(1,),
        in_specs=[
            pl.BlockSpec((_B, 32), lambda i: (0, 0)),
            pl.BlockSpec((_B, 16), lambda i: (0, 0)),
            pl.BlockSpec((64, 32), lambda i: (0, 0)),
            pl.BlockSpec((32, 16), lambda i: (0, 0)),
            pl.BlockSpec((8, _F), lambda i: (0, 0)),
        ],
        out_specs=[
            pl.BlockSpec((_B, 64), lambda i: (0, 0)),
            pl.BlockSpec((_B, 32), lambda i: (0, 0)),
            pl.BlockSpec((8, _F), lambda i: (0, 0)),
        ],
        out_shape=[
            jax.ShapeDtypeStruct((_B, 64), jnp.float32),
            jax.ShapeDtypeStruct((_B, 32), jnp.float32),
            jax.ShapeDtypeStruct((8, _F), jnp.float32),
        ],
    )(edge_x, external_x, we, wx, vb)


_DB = 256
_TPAD = 384
_HID = 256


def _d1_body(om_ref, oh_ref, dm_ref, dh_ref, pe_ref, px_ref, sv_ref, bv_ref,
             w_ref, h_ref, st_ref):
    i = pl.program_id(0)
    ri = lax.broadcasted_iota(jnp.int32, (_DB, 1), 0) + i * _DB
    use_h = ri < 176
    o_raw = jnp.where(use_h, oh_ref[...], om_ref[...])
    d_raw = jnp.where(use_h, dh_ref[...], dm_ref[...])
    tot = jnp.concatenate(
        [o_raw, d_raw, pe_ref[...], px_ref[...],
         jnp.zeros((_DB, 32), jnp.float32)], axis=1)
    scale = sv_ref[0, :]
    shift = sv_ref[1, :]
    arow = sv_ref[2, :]
    y0 = tot * scale[None, :] + shift[None, :]
    y = jnp.where(y0 >= 0.0, y0, arow[None, :] * y0)
    h = lax.dot_general(y, w_ref[...], (((1,), (1,)), ((), ())),
                        preferred_element_type=jnp.float32)
    h = h + bv_ref[0:1, :]
    h_ref[...] = h

    @pl.when(i == 0)
    def _():
        st_ref[...] = jnp.zeros((8, _HID), jnp.float32)

    st_ref[0:1, :] += jnp.sum(h, axis=0, keepdims=True)
    st_ref[1:2, :] += jnp.sum(h * h, axis=0, keepdims=True)


def _d1(om, oh, dm, dh, pe, px, sv, bv, w):
    return pl.pallas_call(
        _d1_body,
        grid=(_B // _DB,),
        in_specs=[
            pl.BlockSpec((_DB, _F), lambda i: (i, 0)),
            pl.BlockSpec((_DB, _F), lambda i: (0, 0)),
            pl.BlockSpec((_DB, _F), lambda i: (i, 0)),
            pl.BlockSpec((_DB, _F), lambda i: (0, 0)),
            pl.BlockSpec((_DB, 64), lambda i: (i, 0)),
            pl.BlockSpec((_DB, 32), lambda i: (i, 0)),
            pl.BlockSpec((8, _TPAD), lambda i: (0, 0)),
            pl.BlockSpec((8, _HID), lambda i: (0, 0)),
            pl.BlockSpec((_HID, _TPAD), lambda i: (0, 0)),
        ],
        out_specs=[
            pl.BlockSpec((_DB, _HID), lambda i: (i, 0)),
            pl.BlockSpec((8, _HID), lambda i: (0, 0)),
        ],
        out_shape=[
            jax.ShapeDtypeStruct((_B, _HID), jnp.float32),
            jax.ShapeDtypeStruct((8, _HID), jnp.float32),
        ],
    )(om, oh, dm, dh, pe, px, sv, bv, w)


def _d2_body(h_ref, v_ref, wo_ref, o_ref):
    scale = v_ref[0, :]
    shift = v_ref[1, :]
    arow = v_ref[2, :]
    y0 = h_ref[...] * scale[None, :] + shift[None, :]
    y = jnp.where(y0 >= 0.0, y0, arow[None, :] * y0)
    o_ref[...] = lax.dot_general(y, wo_ref[...], (((1,), (1,)), ((), ())),
                                 preferred_element_type=jnp.float32) + v_ref[3, 0]


def _d2(h, v, wo):
    return pl.pallas_call(
        _d2_body,
        grid=(8,),
        in_specs=[
            pl.BlockSpec((512, _HID), lambda i: (i, 0)),
            pl.BlockSpec((8, _HID), lambda i: (0, 0)),
            pl.BlockSpec((8, _HID), lambda i: (0, 0)),
        ],
        out_specs=pl.BlockSpec((512, 8), lambda i: (i, 0)),
        out_shape=jax.ShapeDtypeStruct((_B, 8), jnp.float32),
    )(h, v, wo)


def _vec8(*rows):
    n = rows[0].shape[0]
    out = [r[None, :] for r in rows]
    out.append(jnp.zeros((8 - len(rows), n), jnp.float32))
    return jnp.concatenate(out, axis=0)


def _affine(st, g, bb, bias, n):
    c = g.shape[0]
    m = st[0, :c] / n
    v = st[1, :c] / n - m * m
    scale = g * lax.rsqrt(v + 1e-5)
    shift = (bias - m) * scale + bb
    return scale, shift


def _grid_chain(x, counts, p1, p2):
    one = jnp.ones((_F,), jnp.float32)
    zero = jnp.zeros((_F,), jnp.float32)
    h1, st1, _ = _gat_pass(x, x, p1['W'],
                           _vec8(one, zero, one, p1['bias']))
    hp1 = jnp.concatenate(
        [jnp.zeros((24, _F), jnp.float32), h1[:_HPR - 24]], axis=0)
    agg1, _, adj1 = _band(counts, hp1,
                          _vec8(p1['att_src'], p1['att_dst'], p1['bias']))
    sc1, sh1 = _affine(st1 + adj1, p1['bn_g'], p1['bn_b'], p1['bias'], _N)
    a1 = jnp.full((_F,), p1['a'], jnp.float32)
    h2, st2, centm = _gat_pass(h1, agg1, p2['W'],
                               _vec8(sc1, sh1, a1, p2['bias']))
    hp2 = jnp.concatenate(
        [jnp.zeros((24, _F), jnp.float32), h2[:_HPR - 24]], axis=0)
    _, centh, adj2 = _band(counts, hp2,
                           _vec8(p2['att_src'], p2['att_dst'], p2['bias']))
    sc2, sh2 = _affine(st2 + adj2, p2['bn_g'], p2['bn_b'], p2['bias'], _N)
    centh = jnp.concatenate(
        [centh, jnp.zeros((_DB - _KBLK * 8, _F), jnp.float32)], axis=0)
    return centm, centh, sc2, sh2


def kernel(o_grid_x, d_grid_x, edge_index, edge_x, external_x, params):
    counts = _sc_band_counts(edge_index.reshape(-1)).reshape(_CROWS, _CPAD)
    p1, p2 = params['gat']

    om, oh, sco, sho = _grid_chain(o_grid_x.reshape(_N, _F), counts, p1, p2)
    dm, dh, scd, shd = _grid_chain(d_grid_x.reshape(_N, _F), counts, p1, p2)

    pe_p = params['edge']
    px_p = params['ext']
    vb = _vec8(jnp.concatenate([pe_p['b'], jnp.zeros((64,), jnp.float32)]),
               jnp.concatenate([px_p['b'], jnp.zeros((96,), jnp.float32)]))
    pe, px, st0 = _d0(edge_x, external_x, pe_p['W'], px_p['W'], vb)
    sce, she = _affine(st0[0:2], pe_p['bn_g'], pe_p['bn_b'],
                       jnp.zeros((64,), jnp.float32), _B)
    scx, shx = _affine(st0[2:4], px_p['bn_g'], px_p['bn_b'],
                       jnp.zeros((32,), jnp.float32), _B)

    a2 = jnp.full((_F,), p2['a'], jnp.float32)
    z32 = jnp.zeros((32,), jnp.float32)
    scale_t = jnp.concatenate([sco, scd, sce, scx, z32])
    shift_t = jnp.concatenate([sho, shd, she, shx, z32])
    a_t = jnp.concatenate([a2, a2,
                           jnp.full((64,), pe_p['a'], jnp.float32),
                           jnp.full((32,), px_p['a'], jnp.float32),
                           jnp.ones((32,), jnp.float32)])
    p_t = params['tot1']
    w1p = jnp.concatenate(
        [p_t['W'], jnp.zeros((_HID, _TPAD - 352), jnp.float32)], axis=1)
    h, sth = _d1(om, oh, dm, dh, pe, px,
                 _vec8(scale_t, shift_t, a_t), _vec8(p_t['b']), w1p)
    sch, shh = _affine(sth, p_t['bn_g'], p_t['bn_b'],
                       jnp.zeros((_HID,), jnp.float32), _B)

    p_o = params['out']
    a_h = jnp.full((_HID,), p_t['a'], jnp.float32)
    bo_row = jnp.full((_HID,), p_o['b'][0], jnp.float32)
    wo = jnp.concatenate([p_o['W'], jnp.zeros((7, _HID), jnp.float32)], axis=0)
    out = _d2(h, _vec8(sch, shh, a_h, bo_row), wo)
    return out[:, 0]


# speedup vs baseline: 98.2695x; 1.0849x over previous
import functools

import jax
import jax.numpy as jnp
from jax import lax
from jax.experimental import pallas as pl
from jax.experimental.pallas import tpu as pltpu
from jax.experimental.pallas import tpu_sc as plsc

_B = 4096
_GRID = 25
_EPER = 48
_N = _B * _GRID
_F = 128

_RB = 1600
_NBLK = _N // _RB
_RBB = 200
_KBLK = 24
_HEAD = _KBLK * _RBB
_KH = _HEAD // _RB
_CPAD = 256
_WIN = _CPAD
_HPR = (_KBLK - 1) * _RBB + _WIN

_NW = 32
_RPW = 150
_CROWS = _NW * _RPW
_CH = 174
_EB = _CH * 2 * _EPER
_PRIV = _RPW * _CPAD


@functools.partial(
    pl.kernel,
    out_type=jax.ShapeDtypeStruct((_CROWS * _CPAD,), jnp.float32),
    mesh=plsc.VectorSubcoreMesh(core_axis_name="c", subcore_axis_name="s",
                                num_cores=2, num_subcores=16),
    scratch_types=[
        pltpu.VMEM((_EB,), jnp.int32),
        pltpu.VMEM((_PRIV,), jnp.float32),
    ],
    compiler_params=pltpu.CompilerParams(needs_layout_passes=False),
)
def _sc_band_counts(edge_ref, out_ref, edge_v, priv_v):
    cid = lax.axis_index("c")
    sid = lax.axis_index("s")
    w = sid * 2 + cid
    row0 = w * _RPW
    bstart = jnp.clip(row0 - 24, 0, _B - _CH)
    pltpu.sync_copy(edge_ref.at[pl.ds(bstart * (2 * _EPER), _EB)], edge_v)

    z16 = jnp.zeros((16,), jnp.float32)

    def _zero(i, c):
        priv_v[pl.ds(i * 16, 16)] = z16
        return c

    lax.fori_loop(0, _PRIV // 16, _zero, 0)

    lane = lax.iota(jnp.int32, 16)
    ones = jnp.ones((16,), jnp.float32)

    def _batch(bi, c):
        b = bstart + bi
        base = bi * (2 * _EPER)
        for j in range(_EPER // 16):
            sv = edge_v[pl.ds(base + j * 16, 16)]
            dv = edge_v[pl.ds(base + _EPER + j * 16, 16)]
            n = dv + b
            lrow = n - row0
            valid = (lrow >= 0) & (lrow < _RPW)
            blk = lax.div(n, _RBB)
            jcol = (n - blk * _RBB) + (sv - dv) + 24
            flat = jnp.where(valid, lrow * _CPAD + jcol, 0)
            for t in range(16):
                plsc.addupdate_scatter(priv_v, [flat], ones,
                                       mask=valid & (lane == t))
        return c

    lax.fori_loop(0, _CH, _batch, 0)
    pltpu.sync_copy(priv_v, out_ref.at[pl.ds(w * _PRIV, _PRIV)])


def _pass_body(xm_ref, xh_ref, w_ref, vec_ref, h_ref, st_ref, cent_ref):
    i = pl.program_id(0)
    x = jnp.where(i < _KH, xh_ref[...], xm_ref[...])
    scale = vec_ref[0, :]
    shift = vec_ref[1, :]
    arow = vec_ref[2, :]
    bias = vec_ref[3, :]
    y0 = x * scale[None, :] + shift[None, :]
    y = jnp.where(y0 >= 0.0, y0, arow[None, :] * y0)
    h = lax.dot_general(y, w_ref[...], (((1,), (1,)), ((), ())),
                        preferred_element_type=jnp.float32)
    h_ref[...] = h
    t = h + bias[None, :]

    @pl.when(i == 0)
    def _():
        st_ref[...] = jnp.zeros((8, _F), jnp.float32)

    st_ref[0:1, :] += jnp.sum(t, axis=0, keepdims=True)
    st_ref[1:2, :] += jnp.sum(t * t, axis=0, keepdims=True)
    cent_ref[...] = jnp.concatenate(
        [h[12 + 25 * q: 13 + 25 * q, :] for q in range(_RB // 25)], axis=0)


def _pass1_body(x3_ref, vec_ref, w_ref, h_ref, st_ref):
    i = pl.program_id(0)
    x = x3_ref[...].reshape(_RB, _F)
    bias = vec_ref[3, :]
    h = lax.dot_general(x, w_ref[...], (((1,), (1,)), ((), ())),
                        preferred_element_type=jnp.float32)
    h_ref[...] = h
    t = h + bias[None, :]

    @pl.when(i == 0)
    def _():
        st_ref[...] = jnp.zeros((8, _F), jnp.float32)

    st_ref[0:1, :] += jnp.sum(t, axis=0, keepdims=True)
    st_ref[1:2, :] += jnp.sum(t * t, axis=0, keepdims=True)


def _pass1(x3, w, vec):
    return pl.pallas_call(
        _pass1_body,
        grid=(_NBLK,),
        in_specs=[
            pl.BlockSpec((_RB // _GRID, _GRID, _F), lambda i: (i, 0, 0)),
            pl.BlockSpec((8, _F), lambda i: (0, 0)),
            pl.BlockSpec((_F, _F), lambda i: (0, 0)),
        ],
        out_specs=[
            pl.BlockSpec((_RB, _F), lambda i: (i, 0)),
            pl.BlockSpec((8, _F), lambda i: (0, 0)),
        ],
        out_shape=[
            jax.ShapeDtypeStruct((_N, _F), jnp.float32),
            jax.ShapeDtypeStruct((8, _F), jnp.float32),
        ],
    )(x3, vec, w)


def _gat_pass(xm, xh, w, vec):
    return pl.pallas_call(
        _pass_body,
        grid=(_NBLK,),
        in_specs=[
            pl.BlockSpec((_RB, _F), lambda i: (i, 0)),
            pl.BlockSpec((_RB, _F), lambda i: (jnp.minimum(i, _KH - 1), 0)),
            pl.BlockSpec((_F, _F), lambda i: (0, 0)),
            pl.BlockSpec((8, _F), lambda i: (0, 0)),
        ],
        out_specs=[
            pl.BlockSpec((_RB, _F), lambda i: (i, 0)),
            pl.BlockSpec((8, _F), lambda i: (0, 0)),
            pl.BlockSpec((_RB // 25, _F), lambda i: (i, 0)),
        ],
        out_shape=[
            jax.ShapeDtypeStruct((_N, _F), jnp.float32),
            jax.ShapeDtypeStruct((8, _F), jnp.float32),
            jax.ShapeDtypeStruct((_B, _F), jnp.float32),
        ],
    )(xm, xh, w, vec)


def _band_body(cnt_ref, hp_ref, vec_ref, agg_ref, cent_ref, adj_ref):
    k = pl.program_id(0)
    hwin = hp_ref[pl.ds(k * _RBB, _WIN), :]
    att_s = vec_ref[0:1, :]
    att_d = vec_ref[1:2, :]
    bias = vec_ref[2, :]
    asw = lax.dot_general(att_s, hwin, (((1,), (1,)), ((), ())),
                          preferred_element_type=jnp.float32)
    hblk = hwin[24:24 + _RBB, :]
    ad = lax.dot_general(hblk, att_d, (((1,), (1,)), ((), ())),
                         preferred_element_type=jnp.float32)
    al = asw + ad
    al = jnp.where(al >= 0.0, al, 0.2 * al)
    cnt = cnt_ref[...]
    ri = lax.broadcasted_iota(jnp.int32, (_RBB, _WIN), 0)
    ci = lax.broadcasted_iota(jnp.int32, (_RBB, _WIN), 1)
    cnt = cnt + jnp.where(ci == ri + 24, 1.0, 0.0)
    pos = cnt > 0.0
    amax = jnp.max(jnp.where(pos, al, -1e30), axis=1, keepdims=True)
    e = cnt * jnp.exp(al - amax)
    den = jnp.sum(e, axis=1, keepdims=True)
    agg = lax.dot_general(e, hwin, (((1,), (0,)), ((), ())),
                          preferred_element_type=jnp.float32)
    agg = agg * (1.0 / (den + 1e-16))
    agg_ref[...] = agg
    cent_ref[...] = jnp.concatenate(
        [agg[12 + 25 * q: 13 + 25 * q, :] for q in range(8)], axis=0)
    ta = agg + bias[None, :]
    th = hblk + bias[None, :]

    @pl.when(k == 0)
    def _():
        adj_ref[...] = jnp.zeros((8, _F), jnp.float32)

    adj_ref[0:1, :] += (jnp.sum(ta, axis=0, keepdims=True)
                        - jnp.sum(th, axis=0, keepdims=True))
    adj_ref[1:2, :] += (jnp.sum(ta * ta, axis=0, keepdims=True)
                        - jnp.sum(th * th, axis=0, keepdims=True))


def _band(counts, hpad, vec):
    return pl.pallas_call(
        _band_body,
        grid=(_KBLK,),
        in_specs=[
            pl.BlockSpec((_RBB, _CPAD), lambda k: (k, 0)),
            pl.BlockSpec((_HPR, _F), lambda k: (0, 0)),
            pl.BlockSpec((8, _F), lambda k: (0, 0)),
        ],
        out_specs=[
            pl.BlockSpec((_RBB, _F), lambda k: (k, 0)),
            pl.BlockSpec((8, _F), lambda k: (k, 0)),
            pl.BlockSpec((8, _F), lambda k: (0, 0)),
        ],
        out_shape=[
            jax.ShapeDtypeStruct((_HEAD, _F), jnp.float32),
            jax.ShapeDtypeStruct((_KBLK * 8, _F), jnp.float32),
            jax.ShapeDtypeStruct((8, _F), jnp.float32),
        ],
    )(counts, hpad, vec)


def _d0_body(ex_ref, xx_ref, we_ref, wx_ref, vb_ref, pe_ref, px_ref, st_ref):
    pe = lax.dot_general(ex_ref[...], we_ref[...], (((1,), (1,)), ((), ())),
                         preferred_element_type=jnp.float32)
    pe = pe + vb_ref[0:1, :64]
    px = lax.dot_general(xx_ref[...], wx_ref[...], (((1,), (1,)), ((), ())),
                         preferred_element_type=jnp.float32)
    px = px + vb_ref[1:2, :32]
    pe_ref[...] = pe
    px_ref[...] = px
    z64 = jnp.zeros((1, 64), jnp.float32)
    z96 = jnp.zeros((1, 96), jnp.float32)
    st_ref[...] = jnp.concatenate([
        jnp.concatenate([jnp.sum(pe, 0, keepdims=True), z64], axis=1),
        jnp.concatenate([jnp.sum(pe * pe, 0, keepdims=True), z64], axis=1),
        jnp.concatenate([jnp.sum(px, 0, keepdims=True), z96], axis=1),
        jnp.concatenate([jnp.sum(px * px, 0, keepdims=True), z96], axis=1),
        jnp.zeros((4, _F), jnp.float32),
    ], axis=0)


def _d0(edge_x, external_x, we, wx, vb):
    return pl.pallas_call(
        _d0_body,
        grid=(1,),
        in_specs=[
            pl.BlockSpec((_B, 32), lambda i: (0, 0)),
            pl.BlockSpec((_B, 16), lambda i: (0, 0)),
            pl.BlockSpec((64, 32), lambda i: (0, 0)),
            pl.BlockSpec((32, 16), lambda i: (0, 0)),
            pl.BlockSpec((8, _F), lambda i: (0, 0)),
        ],
        out_specs=[
            pl.BlockSpec((_B, 64), lambda i: (0, 0)),
            pl.BlockSpec((_B, 32), lambda i: (0, 0)),
            pl.BlockSpec((8, _F), lambda i: (0, 0)),
        ],
        out_shape=[
            jax.ShapeDtypeStruct((_B, 64), jnp.float32),
            jax.ShapeDtypeStruct((_B, 32), jnp.float32),
            jax.ShapeDtypeStruct((8, _F), jnp.float32),
        ],
    )(edge_x, external_x, we, wx, vb)


_DB = 256
_TPAD = 384
_HID = 256


def _d1_body(om_ref, oh_ref, dm_ref, dh_ref, pe_ref, px_ref, sv_ref, bv_ref,
             w_ref, h_ref, st_ref):
    i = pl.program_id(0)
    ri = lax.broadcasted_iota(jnp.int32, (_DB, 1), 0) + i * _DB
    use_h = ri < 176
    o_raw = jnp.where(use_h, oh_ref[...], om_ref[...])
    d_raw = jnp.where(use_h, dh_ref[...], dm_ref[...])
    tot = jnp.concatenate(
        [o_raw, d_raw, pe_ref[...], px_ref[...],
         jnp.zeros((_DB, 32), jnp.float32)], axis=1)
    scale = sv_ref[0, :]
    shift = sv_ref[1, :]
    arow = sv_ref[2, :]
    y0 = tot * scale[None, :] + shift[None, :]
    y = jnp.where(y0 >= 0.0, y0, arow[None, :] * y0)
    h = lax.dot_general(y, w_ref[...], (((1,), (1,)), ((), ())),
                        preferred_element_type=jnp.float32)
    h = h + bv_ref[0:1, :]
    h_ref[...] = h

    @pl.when(i == 0)
    def _():
        st_ref[...] = jnp.zeros((8, _HID), jnp.float32)

    st_ref[0:1, :] += jnp.sum(h, axis=0, keepdims=True)
    st_ref[1:2, :] += jnp.sum(h * h, axis=0, keepdims=True)


def _d1(om, oh, dm, dh, pe, px, sv, bv, w):
    return pl.pallas_call(
        _d1_body,
        grid=(_B // _DB,),
        in_specs=[
            pl.BlockSpec((_DB, _F), lambda i: (i, 0)),
            pl.BlockSpec((_DB, _F), lambda i: (0, 0)),
            pl.BlockSpec((_DB, _F), lambda i: (i, 0)),
            pl.BlockSpec((_DB, _F), lambda i: (0, 0)),
            pl.BlockSpec((_DB, 64), lambda i: (i, 0)),
            pl.BlockSpec((_DB, 32), lambda i: (i, 0)),
            pl.BlockSpec((8, _TPAD), lambda i: (0, 0)),
            pl.BlockSpec((8, _HID), lambda i: (0, 0)),
            pl.BlockSpec((_HID, _TPAD), lambda i: (0, 0)),
        ],
        out_specs=[
            pl.BlockSpec((_DB, _HID), lambda i: (i, 0)),
            pl.BlockSpec((8, _HID), lambda i: (0, 0)),
        ],
        out_shape=[
            jax.ShapeDtypeStruct((_B, _HID), jnp.float32),
            jax.ShapeDtypeStruct((8, _HID), jnp.float32),
        ],
    )(om, oh, dm, dh, pe, px, sv, bv, w)


def _d2_body(h_ref, v_ref, wo_ref, o_ref):
    scale = v_ref[0, :]
    shift = v_ref[1, :]
    arow = v_ref[2, :]
    y0 = h_ref[...] * scale[None, :] + shift[None, :]
    y = jnp.where(y0 >= 0.0, y0, arow[None, :] * y0)
    o_ref[...] = lax.dot_general(y, wo_ref[...], (((1,), (1,)), ((), ())),
                                 preferred_element_type=jnp.float32) + v_ref[3, 0]


def _d2(h, v, wo):
    return pl.pallas_call(
        _d2_body,
        grid=(8,),
        in_specs=[
            pl.BlockSpec((512, _HID), lambda i: (i, 0)),
            pl.BlockSpec((8, _HID), lambda i: (0, 0)),
            pl.BlockSpec((8, _HID), lambda i: (0, 0)),
        ],
        out_specs=pl.BlockSpec((512, 8), lambda i: (i, 0)),
        out_shape=jax.ShapeDtypeStruct((_B, 8), jnp.float32),
    )(h, v, wo)


def _vec8(*rows):
    n = rows[0].shape[0]
    out = [r[None, :] for r in rows]
    out.append(jnp.zeros((8 - len(rows), n), jnp.float32))
    return jnp.concatenate(out, axis=0)


def _affine(st, g, bb, bias, n):
    c = g.shape[0]
    m = st[0, :c] / n
    v = st[1, :c] / n - m * m
    scale = g * lax.rsqrt(v + 1e-5)
    shift = (bias - m) * scale + bb
    return scale, shift


def _grid_chain(x3, counts, p1, p2):
    one = jnp.ones((_F,), jnp.float32)
    zero = jnp.zeros((_F,), jnp.float32)
    h1, st1 = _pass1(x3, p1['W'], _vec8(one, zero, one, p1['bias']))
    hp1 = jnp.concatenate(
        [jnp.zeros((24, _F), jnp.float32), h1[:_HPR - 24]], axis=0)
    agg1, _, adj1 = _band(counts, hp1,
                          _vec8(p1['att_src'], p1['att_dst'], p1['bias']))
    sc1, sh1 = _affine(st1 + adj1, p1['bn_g'], p1['bn_b'], p1['bias'], _N)
    a1 = jnp.full((_F,), p1['a'], jnp.float32)
    h2, st2, centm = _gat_pass(h1, agg1, p2['W'],
                               _vec8(sc1, sh1, a1, p2['bias']))
    hp2 = jnp.concatenate(
        [jnp.zeros((24, _F), jnp.float32), h2[:_HPR - 24]], axis=0)
    _, centh, adj2 = _band(counts, hp2,
                           _vec8(p2['att_src'], p2['att_dst'], p2['bias']))
    sc2, sh2 = _affine(st2 + adj2, p2['bn_g'], p2['bn_b'], p2['bias'], _N)
    centh = jnp.concatenate(
        [centh, jnp.zeros((_DB - _KBLK * 8, _F), jnp.float32)], axis=0)
    return centm, centh, sc2, sh2


def kernel(o_grid_x, d_grid_x, edge_index, edge_x, external_x, params):
    counts = _sc_band_counts(edge_index.reshape(-1)).reshape(_CROWS, _CPAD)
    p1, p2 = params['gat']

    om, oh, sco, sho = _grid_chain(o_grid_x, counts, p1, p2)
    dm, dh, scd, shd = _grid_chain(d_grid_x, counts, p1, p2)

    pe_p = params['edge']
    px_p = params['ext']
    vb = _vec8(jnp.concatenate([pe_p['b'], jnp.zeros((64,), jnp.float32)]),
               jnp.concatenate([px_p['b'], jnp.zeros((96,), jnp.float32)]))
    pe, px, st0 = _d0(edge_x, external_x, pe_p['W'], px_p['W'], vb)
    sce, she = _affine(st0[0:2], pe_p['bn_g'], pe_p['bn_b'],
                       jnp.zeros((64,), jnp.float32), _B)
    scx, shx = _affine(st0[2:4], px_p['bn_g'], px_p['bn_b'],
                       jnp.zeros((32,), jnp.float32), _B)

    a2 = jnp.full((_F,), p2['a'], jnp.float32)
    z32 = jnp.zeros((32,), jnp.float32)
    scale_t = jnp.concatenate([sco, scd, sce, scx, z32])
    shift_t = jnp.concatenate([sho, shd, she, shx, z32])
    a_t = jnp.concatenate([a2, a2,
                           jnp.full((64,), pe_p['a'], jnp.float32),
                           jnp.full((32,), px_p['a'], jnp.float32),
                           jnp.ones((32,), jnp.float32)])
    p_t = params['tot1']
    w1p = jnp.concatenate(
        [p_t['W'], jnp.zeros((_HID, _TPAD - 352), jnp.float32)], axis=1)
    h, sth = _d1(om, oh, dm, dh, pe, px,
                 _vec8(scale_t, shift_t, a_t), _vec8(p_t['b']), w1p)
    sch, shh = _affine(sth, p_t['bn_g'], p_t['bn_b'],
                       jnp.zeros((_HID,), jnp.float32), _B)

    p_o = params['out']
    a_h = jnp.full((_HID,), p_t['a'], jnp.float32)
    bo_row = jnp.full((_HID,), p_o['b'][0], jnp.float32)
    wo = jnp.concatenate([p_o['W'], jnp.zeros((7, _HID), jnp.float32)], axis=0)
    out = _d2(h, _vec8(sch, shh, a_h, bo_row), wo)
    return out[:, 0]


# speedup vs baseline: 105.3556x; 1.0721x over previous
import functools

import jax
import jax.numpy as jnp
from jax import lax
from jax.experimental import pallas as pl
from jax.experimental.pallas import tpu as pltpu
from jax.experimental.pallas import tpu_sc as plsc

_B = 4096
_GRID = 25
_EPER = 48
_N = _B * _GRID
_F = 128

_RB = 1600
_NBLK = _N // _RB
_RBB = 200
_KBLK = 24
_HEAD = _KBLK * _RBB
_KH = _HEAD // _RB
_CPAD = 256
_WIN = _CPAD
_HPR = (_KBLK - 1) * _RBB + _WIN

_NW = 32
_RPW = 150
_CROWS = _NW * _RPW
_CH = 174
_EB = _CH * 2 * _EPER
_PRIV = _RPW * _CPAD


@functools.partial(
    pl.kernel,
    out_type=jax.ShapeDtypeStruct((_CROWS * _CPAD,), jnp.float32),
    mesh=plsc.VectorSubcoreMesh(core_axis_name="c", subcore_axis_name="s",
                                num_cores=2, num_subcores=16),
    scratch_types=[
        pltpu.VMEM((_EB,), jnp.int32),
        pltpu.VMEM((_PRIV,), jnp.float32),
    ],
    compiler_params=pltpu.CompilerParams(needs_layout_passes=False),
)
def _sc_band_counts(edge_ref, out_ref, edge_v, priv_v):
    cid = lax.axis_index("c")
    sid = lax.axis_index("s")
    w = sid * 2 + cid
    row0 = w * _RPW
    bstart = jnp.clip(row0 - 24, 0, _B - _CH)
    pltpu.sync_copy(edge_ref.at[pl.ds(bstart * (2 * _EPER), _EB)], edge_v)

    z16 = jnp.zeros((16,), jnp.float32)

    def _zero(i, c):
        priv_v[pl.ds(i * 16, 16)] = z16
        return c

    lax.fori_loop(0, _PRIV // 16, _zero, 0)

    lane = lax.iota(jnp.int32, 16)
    ones = jnp.ones((16,), jnp.float32)

    def _batch(bi, c):
        b = bstart + bi
        base = bi * (2 * _EPER)
        for j in range(_EPER // 16):
            sv = edge_v[pl.ds(base + j * 16, 16)]
            dv = edge_v[pl.ds(base + _EPER + j * 16, 16)]
            n = dv + b
            lrow = n - row0
            valid = (lrow >= 0) & (lrow < _RPW)
            blk = lax.div(n, _RBB)
            jcol = (n - blk * _RBB) + (sv - dv) + 24
            flat = jnp.where(valid, lrow * _CPAD + jcol, 0)
            for t in range(16):
                plsc.addupdate_scatter(priv_v, [flat], ones,
                                       mask=valid & (lane == t))
        return c

    lax.fori_loop(0, _CH, _batch, 0)
    pltpu.sync_copy(priv_v, out_ref.at[pl.ds(w * _PRIV, _PRIV)])


def _pass_body(xm_ref, xh_ref, w_ref, vec_ref, h_ref, st_ref, cent_ref):
    i = pl.program_id(0)
    x = jnp.where(i < _KH, xh_ref[...], xm_ref[...]).astype(jnp.float32)
    scale = vec_ref[0, :]
    shift = vec_ref[1, :]
    arow = vec_ref[2, :]
    bias = vec_ref[3, :]
    y0 = x * scale[None, :] + shift[None, :]
    y = jnp.where(y0 >= 0.0, y0, arow[None, :] * y0)
    h = lax.dot_general(y, w_ref[...], (((1,), (1,)), ((), ())),
                        preferred_element_type=jnp.float32)
    h_ref[...] = h.astype(jnp.bfloat16)
    t = h + bias[None, :]

    @pl.when(i == 0)
    def _():
        st_ref[...] = jnp.zeros((8, _F), jnp.float32)

    st_ref[0:1, :] += jnp.sum(t, axis=0, keepdims=True)
    st_ref[1:2, :] += jnp.sum(t * t, axis=0, keepdims=True)
    cent_ref[...] = jnp.concatenate(
        [h[12 + 25 * q: 13 + 25 * q, :] for q in range(_RB // 25)], axis=0)


def _pass1_body(x3_ref, vec_ref, w_ref, h_ref, st_ref):
    i = pl.program_id(0)
    x = x3_ref[...].reshape(_RB, _F)
    bias = vec_ref[3, :]
    h = lax.dot_general(x, w_ref[...], (((1,), (1,)), ((), ())),
                        preferred_element_type=jnp.float32)
    h_ref[...] = h.astype(jnp.bfloat16)
    t = h + bias[None, :]

    @pl.when(i == 0)
    def _():
        st_ref[...] = jnp.zeros((8, _F), jnp.float32)

    st_ref[0:1, :] += jnp.sum(t, axis=0, keepdims=True)
    st_ref[1:2, :] += jnp.sum(t * t, axis=0, keepdims=True)


def _pass1(x3, w, vec):
    return pl.pallas_call(
        _pass1_body,
        grid=(_NBLK,),
        in_specs=[
            pl.BlockSpec((_RB // _GRID, _GRID, _F), lambda i: (i, 0, 0)),
            pl.BlockSpec((8, _F), lambda i: (0, 0)),
            pl.BlockSpec((_F, _F), lambda i: (0, 0)),
        ],
        out_specs=[
            pl.BlockSpec((_RB, _F), lambda i: (i, 0)),
            pl.BlockSpec((8, _F), lambda i: (0, 0)),
        ],
        out_shape=[
            jax.ShapeDtypeStruct((_N, _F), jnp.bfloat16),
            jax.ShapeDtypeStruct((8, _F), jnp.float32),
        ],
    )(x3, vec, w)


def _gat_pass(xm, xh, w, vec):
    return pl.pallas_call(
        _pass_body,
        grid=(_NBLK,),
        in_specs=[
            pl.BlockSpec((_RB, _F), lambda i: (i, 0)),
            pl.BlockSpec((_RB, _F), lambda i: (jnp.minimum(i, _KH - 1), 0)),
            pl.BlockSpec((_F, _F), lambda i: (0, 0)),
            pl.BlockSpec((8, _F), lambda i: (0, 0)),
        ],
        out_specs=[
            pl.BlockSpec((_RB, _F), lambda i: (i, 0)),
            pl.BlockSpec((8, _F), lambda i: (0, 0)),
            pl.BlockSpec((_RB // 25, _F), lambda i: (i, 0)),
        ],
        out_shape=[
            jax.ShapeDtypeStruct((_N, _F), jnp.bfloat16),
            jax.ShapeDtypeStruct((8, _F), jnp.float32),
            jax.ShapeDtypeStruct((_B, _F), jnp.float32),
        ],
    )(xm, xh, w, vec)


def _band_body(cnt_ref, hp_ref, vec_ref, agg_ref, cent_ref, adj_ref):
    k = pl.program_id(0)
    hwin = hp_ref[pl.ds(k * _RBB, _WIN), :].astype(jnp.float32)
    att_s = vec_ref[0:1, :]
    att_d = vec_ref[1:2, :]
    bias = vec_ref[2, :]
    asw = lax.dot_general(att_s, hwin, (((1,), (1,)), ((), ())),
                          preferred_element_type=jnp.float32)
    hblk = hwin[24:24 + _RBB, :]
    ad = lax.dot_general(hblk, att_d, (((1,), (1,)), ((), ())),
                         preferred_element_type=jnp.float32)
    al = asw + ad
    al = jnp.where(al >= 0.0, al, 0.2 * al)
    cnt = cnt_ref[...]
    ri = lax.broadcasted_iota(jnp.int32, (_RBB, _WIN), 0)
    ci = lax.broadcasted_iota(jnp.int32, (_RBB, _WIN), 1)
    cnt = cnt + jnp.where(ci == ri + 24, 1.0, 0.0)
    pos = cnt > 0.0
    amax = jnp.max(jnp.where(pos, al, -1e30), axis=1, keepdims=True)
    e = cnt * jnp.exp(al - amax)
    den = jnp.sum(e, axis=1, keepdims=True)
    agg = lax.dot_general(e, hwin, (((1,), (0,)), ((), ())),
                          preferred_element_type=jnp.float32)
    agg = agg * (1.0 / (den + 1e-16))
    agg_ref[...] = agg.astype(jnp.bfloat16)
    cent_ref[...] = jnp.concatenate(
        [agg[12 + 25 * q: 13 + 25 * q, :] for q in range(8)], axis=0)
    ta = agg + bias[None, :]
    th = hblk + bias[None, :]

    @pl.when(k == 0)
    def _():
        adj_ref[...] = jnp.zeros((8, _F), jnp.float32)

    adj_ref[0:1, :] += (jnp.sum(ta, axis=0, keepdims=True)
                        - jnp.sum(th, axis=0, keepdims=True))
    adj_ref[1:2, :] += (jnp.sum(ta * ta, axis=0, keepdims=True)
                        - jnp.sum(th * th, axis=0, keepdims=True))


def _band(counts, hpad, vec):
    return pl.pallas_call(
        _band_body,
        grid=(_KBLK,),
        in_specs=[
            pl.BlockSpec((_RBB, _CPAD), lambda k: (k, 0)),
            pl.BlockSpec((_HPR, _F), lambda k: (0, 0)),
            pl.BlockSpec((8, _F), lambda k: (0, 0)),
        ],
        out_specs=[
            pl.BlockSpec((_RBB, _F), lambda k: (k, 0)),
            pl.BlockSpec((8, _F), lambda k: (k, 0)),
            pl.BlockSpec((8, _F), lambda k: (0, 0)),
        ],
        out_shape=[
            jax.ShapeDtypeStruct((_HEAD, _F), jnp.bfloat16),
            jax.ShapeDtypeStruct((_KBLK * 8, _F), jnp.float32),
            jax.ShapeDtypeStruct((8, _F), jnp.float32),
        ],
    )(counts, hpad, vec)


def _d0_body(ex_ref, xx_ref, we_ref, wx_ref, vb_ref, pe_ref, px_ref, st_ref):
    pe = lax.dot_general(ex_ref[...], we_ref[...], (((1,), (1,)), ((), ())),
                         preferred_element_type=jnp.float32)
    pe = pe + vb_ref[0:1, :64]
    px = lax.dot_general(xx_ref[...], wx_ref[...], (((1,), (1,)), ((), ())),
                         preferred_element_type=jnp.float32)
    px = px + vb_ref[1:2, :32]
    pe_ref[...] = pe
    px_ref[...] = px
    z64 = jnp.zeros((1, 64), jnp.float32)
    z96 = jnp.zeros((1, 96), jnp.float32)
    st_ref[...] = jnp.concatenate([
        jnp.concatenate([jnp.sum(pe, 0, keepdims=True), z64], axis=1),
        jnp.concatenate([jnp.sum(pe * pe, 0, keepdims=True), z64], axis=1),
        jnp.concatenate([jnp.sum(px, 0, keepdims=True), z96], axis=1),
        jnp.concatenate([jnp.sum(px * px, 0, keepdims=True), z96], axis=1),
        jnp.zeros((4, _F), jnp.float32),
    ], axis=0)


def _d0(edge_x, external_x, we, wx, vb):
    return pl.pallas_call(
        _d0_body,
        grid=(1,),
        in_specs=[
            pl.BlockSpec((_B, 32), lambda i: (0, 0)),
            pl.BlockSpec((_B, 16), lambda i: (0, 0)),
            pl.BlockSpec((64, 32), lambda i: (0, 0)),
            pl.BlockSpec((32, 16), lambda i: (0, 0)),
            pl.BlockSpec((8, _F), lambda i: (0, 0)),
        ],
        out_specs=[
            pl.BlockSpec((_B, 64), lambda i: (0, 0)),
            pl.BlockSpec((_B, 32), lambda i: (0, 0)),
            pl.BlockSpec((8, _F), lambda i: (0, 0)),
        ],
        out_shape=[
            jax.ShapeDtypeStruct((_B, 64), jnp.float32),
            jax.ShapeDtypeStruct((_B, 32), jnp.float32),
            jax.ShapeDtypeStruct((8, _F), jnp.float32),
        ],
    )(edge_x, external_x, we, wx, vb)


_DB = 256
_TPAD = 384
_HID = 256


def _d1_body(om_ref, oh_ref, dm_ref, dh_ref, pe_ref, px_ref, sv_ref, bv_ref,
             w_ref, h_ref, st_ref):
    i = pl.program_id(0)
    ri = lax.broadcasted_iota(jnp.int32, (_DB, 1), 0) + i * _DB
    use_h = ri < 176
    o_raw = jnp.where(use_h, oh_ref[...], om_ref[...])
    d_raw = jnp.where(use_h, dh_ref[...], dm_ref[...])
    tot = jnp.concatenate(
        [o_raw, d_raw, pe_ref[...], px_ref[...],
         jnp.zeros((_DB, 32), jnp.float32)], axis=1)
    scale = sv_ref[0, :]
    shift = sv_ref[1, :]
    arow = sv_ref[2, :]
    y0 = tot * scale[None, :] + shift[None, :]
    y = jnp.where(y0 >= 0.0, y0, arow[None, :] * y0)
    h = lax.dot_general(y, w_ref[...], (((1,), (1,)), ((), ())),
                        preferred_element_type=jnp.float32)
    h = h + bv_ref[0:1, :]
    h_ref[...] = h

    @pl.when(i == 0)
    def _():
        st_ref[...] = jnp.zeros((8, _HID), jnp.float32)

    st_ref[0:1, :] += jnp.sum(h, axis=0, keepdims=True)
    st_ref[1:2, :] += jnp.sum(h * h, axis=0, keepdims=True)


def _d1(om, oh, dm, dh, pe, px, sv, bv, w):
    return pl.pallas_call(
        _d1_body,
        grid=(_B // _DB,),
        in_specs=[
            pl.BlockSpec((_DB, _F), lambda i: (i, 0)),
            pl.BlockSpec((_DB, _F), lambda i: (0, 0)),
            pl.BlockSpec((_DB, _F), lambda i: (i, 0)),
            pl.BlockSpec((_DB, _F), lambda i: (0, 0)),
            pl.BlockSpec((_DB, 64), lambda i: (i, 0)),
            pl.BlockSpec((_DB, 32), lambda i: (i, 0)),
            pl.BlockSpec((8, _TPAD), lambda i: (0, 0)),
            pl.BlockSpec((8, _HID), lambda i: (0, 0)),
            pl.BlockSpec((_HID, _TPAD), lambda i: (0, 0)),
        ],
        out_specs=[
            pl.BlockSpec((_DB, _HID), lambda i: (i, 0)),
            pl.BlockSpec((8, _HID), lambda i: (0, 0)),
        ],
        out_shape=[
            jax.ShapeDtypeStruct((_B, _HID), jnp.float32),
            jax.ShapeDtypeStruct((8, _HID), jnp.float32),
        ],
    )(om, oh, dm, dh, pe, px, sv, bv, w)


def _d2_body(h_ref, v_ref, wo_ref, o_ref):
    scale = v_ref[0, :]
    shift = v_ref[1, :]
    arow = v_ref[2, :]
    y0 = h_ref[...] * scale[None, :] + shift[None, :]
    y = jnp.where(y0 >= 0.0, y0, arow[None, :] * y0)
    o_ref[...] = lax.dot_general(y, wo_ref[...], (((1,), (1,)), ((), ())),
                                 preferred_element_type=jnp.float32) + v_ref[3, 0]


def _d2(h, v, wo):
    return pl.pallas_call(
        _d2_body,
        grid=(8,),
        in_specs=[
            pl.BlockSpec((512, _HID), lambda i: (i, 0)),
            pl.BlockSpec((8, _HID), lambda i: (0, 0)),
            pl.BlockSpec((8, _HID), lambda i: (0, 0)),
        ],
        out_specs=pl.BlockSpec((512, 8), lambda i: (i, 0)),
        out_shape=jax.ShapeDtypeStruct((_B, 8), jnp.float32),
    )(h, v, wo)


def _vec8(*rows):
    n = rows[0].shape[0]
    out = [r[None, :] for r in rows]
    out.append(jnp.zeros((8 - len(rows), n), jnp.float32))
    return jnp.concatenate(out, axis=0)


def _affine(st, g, bb, bias, n):
    c = g.shape[0]
    m = st[0, :c] / n
    v = st[1, :c] / n - m * m
    scale = g * lax.rsqrt(v + 1e-5)
    shift = (bias - m) * scale + bb
    return scale, shift


def _grid_chain(x3, counts, p1, p2):
    one = jnp.ones((_F,), jnp.float32)
    zero = jnp.zeros((_F,), jnp.float32)
    h1, st1 = _pass1(x3, p1['W'], _vec8(one, zero, one, p1['bias']))
    hp1 = jnp.concatenate(
        [jnp.zeros((24, _F), jnp.bfloat16), h1[:_HPR - 24]], axis=0)
    agg1, _, adj1 = _band(counts, hp1,
                          _vec8(p1['att_src'], p1['att_dst'], p1['bias']))
    sc1, sh1 = _affine(st1 + adj1, p1['bn_g'], p1['bn_b'], p1['bias'], _N)
    a1 = jnp.full((_F,), p1['a'], jnp.float32)
    h2, st2, centm = _gat_pass(h1, agg1, p2['W'],
                               _vec8(sc1, sh1, a1, p2['bias']))
    hp2 = jnp.concatenate(
        [jnp.zeros((24, _F), jnp.bfloat16), h2[:_HPR - 24]], axis=0)
    _, centh, adj2 = _band(counts, hp2,
                           _vec8(p2['att_src'], p2['att_dst'], p2['bias']))
    sc2, sh2 = _affine(st2 + adj2, p2['bn_g'], p2['bn_b'], p2['bias'], _N)
    centh = jnp.concatenate(
        [centh, jnp.zeros((_DB - _KBLK * 8, _F), jnp.float32)], axis=0)
    return centm, centh, sc2, sh2


def kernel(o_grid_x, d_grid_x, edge_index, edge_x, external_x, params):
    counts = _sc_band_counts(edge_index.reshape(-1)).reshape(_CROWS, _CPAD)
    p1, p2 = params['gat']

    om, oh, sco, sho = _grid_chain(o_grid_x, counts, p1, p2)
    dm, dh, scd, shd = _grid_chain(d_grid_x, counts, p1, p2)

    pe_p = params['edge']
    px_p = params['ext']
    vb = _vec8(jnp.concatenate([pe_p['b'], jnp.zeros((64,), jnp.float32)]),
               jnp.concatenate([px_p['b'], jnp.zeros((96,), jnp.float32)]))
    pe, px, st0 = _d0(edge_x, external_x, pe_p['W'], px_p['W'], vb)
    sce, she = _affine(st0[0:2], pe_p['bn_g'], pe_p['bn_b'],
                       jnp.zeros((64,), jnp.float32), _B)
    scx, shx = _affine(st0[2:4], px_p['bn_g'], px_p['bn_b'],
                       jnp.zeros((32,), jnp.float32), _B)

    a2 = jnp.full((_F,), p2['a'], jnp.float32)
    z32 = jnp.zeros((32,), jnp.float32)
    scale_t = jnp.concatenate([sco, scd, sce, scx, z32])
    shift_t = jnp.concatenate([sho, shd, she, shx, z32])
    a_t = jnp.concatenate([a2, a2,
                           jnp.full((64,), pe_p['a'], jnp.float32),
                           jnp.full((32,), px_p['a'], jnp.float32),
                           jnp.ones((32,), jnp.float32)])
    p_t = params['tot1']
    w1p = jnp.concatenate(
        [p_t['W'], jnp.zeros((_HID, _TPAD - 352), jnp.float32)], axis=1)
    h, sth = _d1(om, oh, dm, dh, pe, px,
                 _vec8(scale_t, shift_t, a_t), _vec8(p_t['b']), w1p)
    sch, shh = _affine(sth, p_t['bn_g'], p_t['bn_b'],
                       jnp.zeros((_HID,), jnp.float32), _B)

    p_o = params['out']
    a_h = jnp.full((_HID,), p_t['a'], jnp.float32)
    bo_row = jnp.full((_HID,), p_o['b'][0], jnp.float32)
    wo = jnp.concatenate([p_o['W'], jnp.zeros((7, _HID), jnp.float32)], axis=0)
    out = _d2(h, _vec8(sch, shh, a_h, bo_row), wo)
    return out[:, 0]
